# Initial kernel scaffold; baseline (speedup 1.0000x reference)
import jax
import jax.numpy as jnp
from jax.experimental import pallas as pl


def kernel(node_features, edge_features, W0, b0, We1, be1, We2, be2, b_conv, gW_ih, gW_hh, gb_ih, gb_hh, lW_ih, lW_hh, lb_ih, lb_hh, edge_index, graph_index):
    raise NotImplementedError("write your pallas kernel here")


# speedup vs baseline: 2.3990x; 2.3990x over previous
import functools

import jax
import jax.numpy as jnp
from jax import lax
from jax.experimental import pallas as pl
from jax.experimental.pallas import tpu as pltpu
from jax.experimental.pallas import tpu_sc as plsc

N = 10000
E = 160000
F_IN = 128
DIM = 32
G = 64

NW = 32
CHUNK = 128
CHUNKS = 40
PER_W = CHUNKS * CHUNK
E_PAD = NW * PER_W
N_PAD = 10240
STRIPE = N_PAD // 16
BE = 512
BN = 1024

_MESH = plsc.VectorSubcoreMesh(
    core_axis_name="c", subcore_axis_name="s", num_cores=2, num_subcores=16)


@functools.partial(
    pl.kernel,
    out_type=jax.ShapeDtypeStruct((E_PAD, DIM), jnp.float32),
    mesh=_MESH,
    scratch_types=[
        pltpu.VMEM((CHUNKS, CHUNK), jnp.int32),
        pltpu.VMEM((CHUNK, DIM), jnp.float32),
    ],
    compiler_params=pltpu.CompilerParams(use_tc_tiling_on_sc=False),
)
def _sc_gather(table, src3, out, idx_v, rows_v):
    cid = lax.axis_index("c")
    sid = lax.axis_index("s")
    w = sid * 2 + cid
    pltpu.sync_copy(src3.at[w], idx_v)

    def step(j, carry):
        pltpu.sync_copy(table.at[idx_v.at[j]], rows_v)
        pltpu.sync_copy(rows_v, out.at[pl.ds(w * PER_W + j * CHUNK, CHUNK)])
        return carry

    lax.fori_loop(0, CHUNKS, step, 0)


@functools.partial(
    pl.kernel,
    out_type=(
        jax.ShapeDtypeStruct((2, N_PAD, DIM), jnp.float32),
        jax.ShapeDtypeStruct((2, N_PAD, 16), jnp.float32),
    ),
    mesh=_MESH,
    scratch_types=[
        pltpu.VMEM((CHUNKS, CHUNK), jnp.int32),
        pltpu.VMEM((CHUNK, DIM), jnp.float32),
        pltpu.VMEM((CHUNK, 16), jnp.float32),
        pltpu.VMEM_SHARED((N_PAD, DIM), jnp.float32),
        pltpu.VMEM_SHARED((N_PAD, 16), jnp.float32),
    ],
    compiler_params=pltpu.CompilerParams(use_tc_tiling_on_sc=False),
)
def _sc_scatter(msg, dst3, zeros32, zeros16, ones16,
                agg_out, cnt_out, idx_v, rows_v, ones_v, acc, cacc):
    cid = lax.axis_index("c")
    sid = lax.axis_index("s")
    w = sid * 2 + cid
    r0 = sid * STRIPE
    pltpu.sync_copy(zeros32, acc.at[pl.ds(r0, STRIPE)])
    pltpu.sync_copy(zeros16, cacc.at[pl.ds(r0, STRIPE)])
    pltpu.sync_copy(ones16, ones_v)
    pltpu.sync_copy(dst3.at[w], idx_v)
    plsc.subcore_barrier()

    def step(j, carry):
        pltpu.sync_copy(msg.at[pl.ds(w * PER_W + j * CHUNK, CHUNK)], rows_v)
        pltpu.sync_copy(rows_v, acc.at[idx_v.at[j]], add=True)
        pltpu.sync_copy(ones_v, cacc.at[idx_v.at[j]], add=True)
        return carry

    lax.fori_loop(0, CHUNKS, step, 0)
    plsc.subcore_barrier()
    pltpu.sync_copy(acc.at[pl.ds(r0, STRIPE)],
                    agg_out.at[cid, pl.ds(r0, STRIPE)])
    pltpu.sync_copy(cacc.at[pl.ds(r0, STRIPE)],
                    cnt_out.at[cid, pl.ds(r0, STRIPE)])


def _lin0_body(nf_ref, w_ref, b_ref, out_ref):
    acc = jnp.dot(nf_ref[...], w_ref[...], preferred_element_type=jnp.float32)
    out_ref[...] = jnp.maximum(acc + b_ref[...], 0.0)


_lin0 = pl.pallas_call(
    _lin0_body,
    grid=(N_PAD // BN,),
    in_specs=[
        pl.BlockSpec((BN, F_IN), lambda i: (i, 0)),
        pl.BlockSpec((F_IN, DIM), lambda i: (0, 0)),
        pl.BlockSpec((1, DIM), lambda i: (0, 0)),
    ],
    out_specs=pl.BlockSpec((BN, DIM), lambda i: (i, 0)),
    out_shape=jax.ShapeDtypeStruct((N_PAD, DIM), jnp.float32),
)


def _msg_body(ef_ref, xg_ref, we1_ref, be1_ref, we2_ref, be2_ref, msg_ref):
    efT = ef_ref[...].T
    h1T = jnp.dot(we1_ref[...], efT, preferred_element_type=jnp.float32)
    h1T = jnp.maximum(h1T + be1_ref[...], 0.0)
    ewT = jnp.dot(we2_ref[...], h1T, preferred_element_type=jnp.float32)
    ewT = ewT + be2_ref[...]
    xgT = xg_ref[...].T
    ew3 = ewT.reshape(DIM, DIM, BE)
    msgT = jnp.sum(ew3 * xgT[:, None, :], axis=0)
    msg_ref[...] = msgT.T


_msg = pl.pallas_call(
    _msg_body,
    grid=(E_PAD // BE,),
    in_specs=[
        pl.BlockSpec((BE, 16), lambda i: (i, 0)),
        pl.BlockSpec((BE, DIM), lambda i: (i, 0)),
        pl.BlockSpec((F_IN, 16), lambda i: (0, 0)),
        pl.BlockSpec((F_IN, 1), lambda i: (0, 0)),
        pl.BlockSpec((DIM * DIM, F_IN), lambda i: (0, 0)),
        pl.BlockSpec((DIM * DIM, 1), lambda i: (0, 0)),
    ],
    out_specs=pl.BlockSpec((BE, DIM), lambda i: (i, 0)),
    out_shape=jax.ShapeDtypeStruct((E_PAD, DIM), jnp.float32),
)


def _gru_body(agg_ref, cnt_ref, h_ref, bconv_ref, wih_ref, whh_ref,
              bih_ref, bhh_ref, out_ref):
    agg = agg_ref[0] + agg_ref[1]
    cnt = jnp.maximum(cnt_ref[0, :, 0:1] + cnt_ref[1, :, 0:1], 1.0)
    m = jnp.maximum(agg / cnt + bconv_ref[...], 0.0)
    h = h_ref[...]
    gi = jnp.dot(m, wih_ref[...], preferred_element_type=jnp.float32)
    gi = gi + bih_ref[...]
    gh = jnp.dot(h, whh_ref[...], preferred_element_type=jnp.float32)
    gh = gh + bhh_ref[...]
    r = jax.nn.sigmoid(gi[:, 0:DIM] + gh[:, 0:DIM])
    z = jax.nn.sigmoid(gi[:, DIM:2 * DIM] + gh[:, DIM:2 * DIM])
    n = jnp.tanh(gi[:, 2 * DIM:] + r * gh[:, 2 * DIM:])
    out_ref[...] = (1.0 - z) * n + z * h


_gru = pl.pallas_call(
    _gru_body,
    grid=(N_PAD // BN,),
    in_specs=[
        pl.BlockSpec((2, BN, DIM), lambda i: (0, i, 0)),
        pl.BlockSpec((2, BN, 16), lambda i: (0, i, 0)),
        pl.BlockSpec((BN, DIM), lambda i: (i, 0)),
        pl.BlockSpec((1, DIM), lambda i: (0, 0)),
        pl.BlockSpec((DIM, 3 * DIM), lambda i: (0, 0)),
        pl.BlockSpec((DIM, 3 * DIM), lambda i: (0, 0)),
        pl.BlockSpec((1, 3 * DIM), lambda i: (0, 0)),
        pl.BlockSpec((1, 3 * DIM), lambda i: (0, 0)),
    ],
    out_specs=pl.BlockSpec((BN, DIM), lambda i: (i, 0)),
    out_shape=jax.ShapeDtypeStruct((N_PAD, DIM), jnp.float32),
)


def _s2s_body(h_ref, gcol_ref, grow_ref, wih_ref, whh_ref, bih_ref, bhh_ref,
              q_ref):
    hfull = h_ref[...]
    gcol = gcol_ref[...]
    grow = grow_ref[...]
    iota_row = lax.broadcasted_iota(jnp.int32, (1, G), 1)
    onehot = (gcol == iota_row).astype(jnp.float32)
    iota_col = lax.broadcasted_iota(jnp.int32, (G, 1), 0)
    onehotT = (iota_col == grow).astype(jnp.float32)
    valid = (gcol < G).astype(jnp.float32)

    q_star = jnp.zeros((G, 2 * DIM), jnp.float32)
    lh = jnp.zeros((G, DIM), jnp.float32)
    lc = jnp.zeros((G, DIM), jnp.float32)
    for _ in range(3):
        gates = (jnp.dot(q_star, wih_ref[...], preferred_element_type=jnp.float32)
                 + bih_ref[...]
                 + jnp.dot(lh, whh_ref[...], preferred_element_type=jnp.float32)
                 + bhh_ref[...])
        i_ = jax.nn.sigmoid(gates[:, 0:DIM])
        f_ = jax.nn.sigmoid(gates[:, DIM:2 * DIM])
        g_ = jnp.tanh(gates[:, 2 * DIM:3 * DIM])
        o_ = jax.nn.sigmoid(gates[:, 3 * DIM:])
        lc = f_ * lc + i_ * g_
        lh = o_ * jnp.tanh(lc)
        q = lh
        qn = jnp.dot(onehot, q, preferred_element_type=jnp.float32)
        e = jnp.sum(hfull * qn, axis=1, keepdims=True)
        e = e * valid
        big = jnp.where(onehot > 0.0, e, -1e30)
        emax = jnp.max(big, axis=0, keepdims=True)
        emax = jnp.where(emax < -1e29, 0.0, emax)
        emax_n = jnp.sum(onehot * emax, axis=1, keepdims=True)
        ee = jnp.exp(e - emax_n) * valid
        denom = jnp.sum(onehot * ee, axis=0, keepdims=True)
        denom_n = jnp.sum(onehot * denom, axis=1, keepdims=True)
        a = ee / (denom_n + 1e-16)
        rvec = jnp.dot(onehotT, a * hfull,
                       preferred_element_type=jnp.float32)
        q_star = jnp.concatenate([q, rvec], axis=1)
    q_ref[...] = q_star


_s2s = pl.pallas_call(
    _s2s_body,
    out_shape=jax.ShapeDtypeStruct((G, 2 * DIM), jnp.float32),
)


def kernel(node_features, edge_features, W0, b0, We1, be1, We2, be2, b_conv,
           gW_ih, gW_hh, gb_ih, gb_hh, lW_ih, lW_hh, lb_ih, lb_hh,
           edge_index, graph_index):
    f32 = jnp.float32
    nf = jnp.zeros((N_PAD, F_IN), f32).at[:N].set(node_features)
    ef = jnp.zeros((E_PAD, 16), f32).at[:E, :11].set(edge_features)
    src3 = (jnp.zeros((E_PAD,), jnp.int32).at[:E].set(edge_index[0])
            .reshape(NW, CHUNKS, CHUNK))
    dst3 = (jnp.full((E_PAD,), N, jnp.int32).at[:E].set(edge_index[1])
            .reshape(NW, CHUNKS, CHUNK))
    gcol = jnp.full((N_PAD, 1), G, jnp.int32).at[:N, 0].set(graph_index)
    grow = gcol.reshape(1, N_PAD)

    W0T = W0.T
    We1p = jnp.zeros((F_IN, 16), f32).at[:, :11].set(We1)
    be1c = be1.reshape(F_IN, 1)
    be2c = be2.reshape(DIM * DIM, 1)
    b0r = b0.reshape(1, DIM)
    bconv = b_conv.reshape(1, DIM)
    gWihT = gW_ih.T
    gWhhT = gW_hh.T
    gbih = gb_ih.reshape(1, 3 * DIM)
    gbhh = gb_hh.reshape(1, 3 * DIM)
    lWihT = lW_ih.T
    lWhhT = lW_hh.T
    lbih = lb_ih.reshape(1, 4 * DIM)
    lbhh = lb_hh.reshape(1, 4 * DIM)

    zeros32 = jnp.zeros((STRIPE, DIM), f32)
    zeros16 = jnp.zeros((STRIPE, 16), f32)
    ones16 = jnp.ones((CHUNK, 16), f32)

    h = _lin0(nf, W0T, b0r)
    for _ in range(3):
        xg = _sc_gather(h, src3)
        msg = _msg(ef, xg, We1p, be1c, We2, be2c)
        aggp, cntp = _sc_scatter(msg, dst3, zeros32, zeros16, ones16)
        h = _gru(aggp, cntp, h, bconv, gWihT, gWhhT, gbih, gbhh)
    q_star = _s2s(h, gcol, grow, lWihT, lWhhT, lbih, lbhh)
    return (q_star, h[:N])


# speedup vs baseline: 2.8993x; 1.2086x over previous
import functools

import jax
import jax.numpy as jnp
from jax import lax
from jax.experimental import pallas as pl
from jax.experimental.pallas import tpu as pltpu
from jax.experimental.pallas import tpu_sc as plsc

N = 10000
E = 160000
F_IN = 128
DIM = 32
G = 64

NW = 32
CHUNK = 128
CHUNKS = 40
PER_W = CHUNKS * CHUNK
E_PAD = NW * PER_W
N_PAD = 10240
STRIPE = N_PAD // 16
BE = 512
BN = 1024

_MESH = plsc.VectorSubcoreMesh(
    core_axis_name="c", subcore_axis_name="s", num_cores=2, num_subcores=16)


NBUF = 4
ROUNDS = CHUNKS // NBUF


@functools.partial(
    pl.kernel,
    out_type=jax.ShapeDtypeStruct((E_PAD, DIM), jnp.float32),
    mesh=_MESH,
    scratch_types=[
        pltpu.VMEM((CHUNKS, CHUNK), jnp.int32),
        pltpu.VMEM((NBUF, CHUNK, DIM), jnp.float32),
        pltpu.VMEM_SHARED((N_PAD, DIM), jnp.float32),
        pltpu.SemaphoreType.DMA,
        pltpu.SemaphoreType.DMA,
    ],
    compiler_params=pltpu.CompilerParams(use_tc_tiling_on_sc=False),
)
def _sc_gather(table, src3, out, idx_v, bufs, tbl, gsem, wsem):
    cid = lax.axis_index("c")
    sid = lax.axis_index("s")
    w = sid * 2 + cid
    r0 = sid * STRIPE
    pltpu.sync_copy(table.at[pl.ds(r0, STRIPE)], tbl.at[pl.ds(r0, STRIPE)])
    pltpu.sync_copy(src3.at[w], idx_v)
    plsc.subcore_barrier()

    def round_(i, carry):
        @pl.when(i > 0)
        def _():
            for k in range(NBUF):
                pltpu.make_async_copy(
                    bufs.at[k], out.at[pl.ds(w * PER_W, CHUNK)], wsem).wait()
        gds = [
            pltpu.async_copy(tbl.at[idx_v.at[i * NBUF + k]], bufs.at[k], gsem)
            for k in range(NBUF)
        ]
        for k in range(NBUF):
            gds[k].wait()
        for k in range(NBUF):
            pltpu.async_copy(
                bufs.at[k],
                out.at[pl.ds(w * PER_W + (i * NBUF + k) * CHUNK, CHUNK)],
                wsem)
        return carry

    lax.fori_loop(0, ROUNDS, round_, 0)
    for k in range(NBUF):
        pltpu.make_async_copy(
            bufs.at[k], out.at[pl.ds(w * PER_W, CHUNK)], wsem).wait()


@functools.partial(
    pl.kernel,
    out_type=jax.ShapeDtypeStruct((2, N_PAD, DIM), jnp.float32),
    mesh=_MESH,
    scratch_types=[
        pltpu.VMEM((CHUNKS, CHUNK), jnp.int32),
        pltpu.VMEM((2 * NBUF, CHUNK, DIM), jnp.float32),
        pltpu.VMEM_SHARED((N_PAD, DIM), jnp.float32),
        pltpu.SemaphoreType.DMA,
    ],
    compiler_params=pltpu.CompilerParams(use_tc_tiling_on_sc=False),
)
def _sc_scatter(msg, dst3, zeros32, agg_out, idx_v, bufs, acc, lsem):
    cid = lax.axis_index("c")
    sid = lax.axis_index("s")
    w = sid * 2 + cid
    r0 = sid * STRIPE
    pltpu.sync_copy(zeros32, acc.at[pl.ds(r0, STRIPE)])
    pltpu.sync_copy(dst3.at[w], idx_v)
    for k in range(NBUF):
        pltpu.async_copy(msg.at[pl.ds((w * CHUNKS + k) * CHUNK, CHUNK)],
                         bufs.at[k], lsem)
    plsc.subcore_barrier()

    def round_(i, carry):
        p = lax.rem(i, 2)
        for k in range(NBUF):
            pltpu.make_async_copy(
                msg.at[pl.ds(w * PER_W, CHUNK)], bufs.at[k], lsem).wait()
        @pl.when(i < ROUNDS - 1)
        def _():
            for k in range(NBUF):
                pltpu.async_copy(
                    msg.at[pl.ds((w * CHUNKS + (i + 1) * NBUF + k) * CHUNK,
                                 CHUNK)],
                    bufs.at[(1 - p) * NBUF + k], lsem)
        for k in range(NBUF):
            pltpu.sync_copy(bufs.at[p * NBUF + k],
                            acc.at[idx_v.at[i * NBUF + k]], add=True)
        return carry

    lax.fori_loop(0, ROUNDS, round_, 0)
    plsc.subcore_barrier()
    pltpu.sync_copy(acc.at[pl.ds(r0, STRIPE)],
                    agg_out.at[cid, pl.ds(r0, STRIPE)])


@functools.partial(
    pl.kernel,
    out_type=jax.ShapeDtypeStruct((2, N_PAD, 16), jnp.float32),
    mesh=_MESH,
    scratch_types=[
        pltpu.VMEM((CHUNKS, CHUNK), jnp.int32),
        pltpu.VMEM((CHUNK, 16), jnp.float32),
        pltpu.VMEM_SHARED((N_PAD, 16), jnp.float32),
    ],
    compiler_params=pltpu.CompilerParams(use_tc_tiling_on_sc=False),
)
def _sc_cnt(dst3, zeros16, ones16, cnt_out, idx_v, ones_v, cacc):
    cid = lax.axis_index("c")
    sid = lax.axis_index("s")
    w = sid * 2 + cid
    r0 = sid * STRIPE
    pltpu.sync_copy(zeros16, cacc.at[pl.ds(r0, STRIPE)])
    pltpu.sync_copy(ones16, ones_v)
    pltpu.sync_copy(dst3.at[w], idx_v)
    plsc.subcore_barrier()

    def step(j, carry):
        pltpu.sync_copy(ones_v, cacc.at[idx_v.at[j]], add=True)
        return carry

    lax.fori_loop(0, CHUNKS, step, 0)
    plsc.subcore_barrier()
    pltpu.sync_copy(cacc.at[pl.ds(r0, STRIPE)],
                    cnt_out.at[cid, pl.ds(r0, STRIPE)])


def _lin0_body(nf_ref, w_ref, b_ref, out_ref):
    acc = jnp.dot(nf_ref[...], w_ref[...], preferred_element_type=jnp.float32)
    out_ref[...] = jnp.maximum(acc + b_ref[...], 0.0)


_lin0 = pl.pallas_call(
    _lin0_body,
    grid=(N_PAD // BN,),
    in_specs=[
        pl.BlockSpec((BN, F_IN), lambda i: (i, 0)),
        pl.BlockSpec((F_IN, DIM), lambda i: (0, 0)),
        pl.BlockSpec((1, DIM), lambda i: (0, 0)),
    ],
    out_specs=pl.BlockSpec((BN, DIM), lambda i: (i, 0)),
    out_shape=jax.ShapeDtypeStruct((N_PAD, DIM), jnp.float32),
)


def _msg_body(ef_ref, xg_ref, we1_ref, be1_ref, we2_ref, be2_ref, msg_ref):
    efT = ef_ref[...].T
    h1T = jnp.dot(we1_ref[...], efT, preferred_element_type=jnp.float32)
    h1T = jnp.maximum(h1T + be1_ref[...], 0.0)
    ewT = jnp.dot(we2_ref[...], h1T, preferred_element_type=jnp.float32)
    ewT = ewT + be2_ref[...]
    xgT = xg_ref[...].T
    ew3 = ewT.reshape(DIM, DIM, BE)
    msgT = jnp.sum(ew3 * xgT[:, None, :], axis=0)
    msg_ref[...] = msgT.T


_msg = pl.pallas_call(
    _msg_body,
    grid=(E_PAD // BE,),
    in_specs=[
        pl.BlockSpec((BE, 16), lambda i: (i, 0)),
        pl.BlockSpec((BE, DIM), lambda i: (i, 0)),
        pl.BlockSpec((F_IN, 16), lambda i: (0, 0)),
        pl.BlockSpec((F_IN, 1), lambda i: (0, 0)),
        pl.BlockSpec((DIM * DIM, F_IN), lambda i: (0, 0)),
        pl.BlockSpec((DIM * DIM, 1), lambda i: (0, 0)),
    ],
    out_specs=pl.BlockSpec((BE, DIM), lambda i: (i, 0)),
    out_shape=jax.ShapeDtypeStruct((E_PAD, DIM), jnp.float32),
)


def _gru_body(agg_ref, cnt_ref, h_ref, bconv_ref, wih_ref, whh_ref,
              bih_ref, bhh_ref, out_ref):
    agg = agg_ref[0] + agg_ref[1]
    cnt = jnp.maximum(cnt_ref[0, :, 0:1] + cnt_ref[1, :, 0:1], 1.0)
    m = jnp.maximum(agg / cnt + bconv_ref[...], 0.0)
    h = h_ref[...]
    gi = jnp.dot(m, wih_ref[...], preferred_element_type=jnp.float32)
    gi = gi + bih_ref[...]
    gh = jnp.dot(h, whh_ref[...], preferred_element_type=jnp.float32)
    gh = gh + bhh_ref[...]
    r = jax.nn.sigmoid(gi[:, 0:DIM] + gh[:, 0:DIM])
    z = jax.nn.sigmoid(gi[:, DIM:2 * DIM] + gh[:, DIM:2 * DIM])
    n = jnp.tanh(gi[:, 2 * DIM:] + r * gh[:, 2 * DIM:])
    out_ref[...] = (1.0 - z) * n + z * h


_gru = pl.pallas_call(
    _gru_body,
    grid=(N_PAD // BN,),
    in_specs=[
        pl.BlockSpec((2, BN, DIM), lambda i: (0, i, 0)),
        pl.BlockSpec((2, BN, 16), lambda i: (0, i, 0)),
        pl.BlockSpec((BN, DIM), lambda i: (i, 0)),
        pl.BlockSpec((1, DIM), lambda i: (0, 0)),
        pl.BlockSpec((DIM, 3 * DIM), lambda i: (0, 0)),
        pl.BlockSpec((DIM, 3 * DIM), lambda i: (0, 0)),
        pl.BlockSpec((1, 3 * DIM), lambda i: (0, 0)),
        pl.BlockSpec((1, 3 * DIM), lambda i: (0, 0)),
    ],
    out_specs=pl.BlockSpec((BN, DIM), lambda i: (i, 0)),
    out_shape=jax.ShapeDtypeStruct((N_PAD, DIM), jnp.float32),
)


def _s2s_body(h_ref, gcol_ref, grow_ref, wih_ref, whh_ref, bih_ref, bhh_ref,
              q_ref):
    hfull = h_ref[...]
    gcol = gcol_ref[...]
    grow = grow_ref[...]
    iota_row = lax.broadcasted_iota(jnp.int32, (1, G), 1)
    onehot = (gcol == iota_row).astype(jnp.float32)
    iota_col = lax.broadcasted_iota(jnp.int32, (G, 1), 0)
    onehotT = (iota_col == grow).astype(jnp.float32)
    valid = (gcol < G).astype(jnp.float32)

    q_star = jnp.zeros((G, 2 * DIM), jnp.float32)
    lh = jnp.zeros((G, DIM), jnp.float32)
    lc = jnp.zeros((G, DIM), jnp.float32)
    for _ in range(3):
        gates = (jnp.dot(q_star, wih_ref[...], preferred_element_type=jnp.float32)
                 + bih_ref[...]
                 + jnp.dot(lh, whh_ref[...], preferred_element_type=jnp.float32)
                 + bhh_ref[...])
        i_ = jax.nn.sigmoid(gates[:, 0:DIM])
        f_ = jax.nn.sigmoid(gates[:, DIM:2 * DIM])
        g_ = jnp.tanh(gates[:, 2 * DIM:3 * DIM])
        o_ = jax.nn.sigmoid(gates[:, 3 * DIM:])
        lc = f_ * lc + i_ * g_
        lh = o_ * jnp.tanh(lc)
        q = lh
        qn = jnp.dot(onehot, q, preferred_element_type=jnp.float32)
        e = jnp.sum(hfull * qn, axis=1, keepdims=True)
        e = e * valid
        big = jnp.where(onehot > 0.0, e, -1e30)
        emax = jnp.max(big, axis=0, keepdims=True)
        emax = jnp.where(emax < -1e29, 0.0, emax)
        emax_n = jnp.sum(onehot * emax, axis=1, keepdims=True)
        ee = jnp.exp(e - emax_n) * valid
        denom = jnp.sum(onehot * ee, axis=0, keepdims=True)
        denom_n = jnp.sum(onehot * denom, axis=1, keepdims=True)
        a = ee / (denom_n + 1e-16)
        rvec = jnp.dot(onehotT, a * hfull,
                       preferred_element_type=jnp.float32)
        q_star = jnp.concatenate([q, rvec], axis=1)
    q_ref[...] = q_star


_s2s = pl.pallas_call(
    _s2s_body,
    out_shape=jax.ShapeDtypeStruct((G, 2 * DIM), jnp.float32),
)


def kernel(node_features, edge_features, W0, b0, We1, be1, We2, be2, b_conv,
           gW_ih, gW_hh, gb_ih, gb_hh, lW_ih, lW_hh, lb_ih, lb_hh,
           edge_index, graph_index):
    f32 = jnp.float32
    nf = jnp.zeros((N_PAD, F_IN), f32).at[:N].set(node_features)
    ef = jnp.zeros((E_PAD, 16), f32).at[:E, :11].set(edge_features)
    src3 = (jnp.zeros((E_PAD,), jnp.int32).at[:E].set(edge_index[0])
            .reshape(NW, CHUNKS, CHUNK))
    dst3 = (jnp.full((E_PAD,), N, jnp.int32).at[:E].set(edge_index[1])
            .reshape(NW, CHUNKS, CHUNK))
    gcol = jnp.full((N_PAD, 1), G, jnp.int32).at[:N, 0].set(graph_index)
    grow = gcol.reshape(1, N_PAD)

    W0T = W0.T
    We1p = jnp.zeros((F_IN, 16), f32).at[:, :11].set(We1)
    be1c = be1.reshape(F_IN, 1)
    be2c = be2.reshape(DIM * DIM, 1)
    b0r = b0.reshape(1, DIM)
    bconv = b_conv.reshape(1, DIM)
    gWihT = gW_ih.T
    gWhhT = gW_hh.T
    gbih = gb_ih.reshape(1, 3 * DIM)
    gbhh = gb_hh.reshape(1, 3 * DIM)
    lWihT = lW_ih.T
    lWhhT = lW_hh.T
    lbih = lb_ih.reshape(1, 4 * DIM)
    lbhh = lb_hh.reshape(1, 4 * DIM)

    zeros32 = jnp.zeros((STRIPE, DIM), f32)
    zeros16 = jnp.zeros((STRIPE, 16), f32)
    ones16 = jnp.ones((CHUNK, 16), f32)

    h = _lin0(nf, W0T, b0r)
    cntp = _sc_cnt(dst3, zeros16, ones16)
    for _ in range(3):
        xg = _sc_gather(h, src3)
        msg = _msg(ef, xg, We1p, be1c, We2, be2c)
        aggp = _sc_scatter(msg, dst3, zeros32)
        h = _gru(aggp, cntp, h, bconv, gWihT, gWhhT, gbih, gbhh)
    q_star = _s2s(h, gcol, grow, lWihT, lWhhT, lbih, lbhh)
    return (q_star, h[:N])


# speedup vs baseline: 3.5927x; 1.2392x over previous
import functools

import jax
import jax.numpy as jnp
from jax import lax
from jax.experimental import pallas as pl
from jax.experimental.pallas import tpu as pltpu
from jax.experimental.pallas import tpu_sc as plsc

N = 10000
E = 160000
F_IN = 128
DIM = 32
G = 64

NW = 32
CHUNK = 128
CHUNKS = 40
PER_W = CHUNKS * CHUNK
E_PAD = NW * PER_W
N_PAD = 10240
STRIPE = N_PAD // 16
BE = 1024
BN = 1024

_MESH = plsc.VectorSubcoreMesh(
    core_axis_name="c", subcore_axis_name="s", num_cores=2, num_subcores=16)


NBUF = 4
ROUNDS = CHUNKS // NBUF


@functools.partial(
    pl.kernel,
    out_type=jax.ShapeDtypeStruct((E_PAD, DIM), jnp.float32),
    mesh=_MESH,
    scratch_types=[
        pltpu.VMEM((CHUNKS, CHUNK), jnp.int32),
        pltpu.VMEM((NBUF, CHUNK, DIM), jnp.float32),
        pltpu.VMEM_SHARED((N_PAD, DIM), jnp.float32),
        pltpu.SemaphoreType.DMA,
        pltpu.SemaphoreType.DMA,
    ],
    compiler_params=pltpu.CompilerParams(use_tc_tiling_on_sc=False),
)
def _sc_gather(table, src3, out, idx_v, bufs, tbl, gsem, wsem):
    cid = lax.axis_index("c")
    sid = lax.axis_index("s")
    w = sid * 2 + cid
    r0 = sid * STRIPE
    pltpu.sync_copy(table.at[pl.ds(r0, STRIPE)], tbl.at[pl.ds(r0, STRIPE)])
    pltpu.sync_copy(src3.at[w], idx_v)
    plsc.subcore_barrier()

    def round_(i, carry):
        @pl.when(i > 0)
        def _():
            for k in range(NBUF):
                pltpu.make_async_copy(
                    bufs.at[k], out.at[pl.ds(w * PER_W, CHUNK)], wsem).wait()
        gds = [
            pltpu.async_copy(tbl.at[idx_v.at[i * NBUF + k]], bufs.at[k], gsem)
            for k in range(NBUF)
        ]
        for k in range(NBUF):
            gds[k].wait()
        for k in range(NBUF):
            pltpu.async_copy(
                bufs.at[k],
                out.at[pl.ds(w * PER_W + (i * NBUF + k) * CHUNK, CHUNK)],
                wsem)
        return carry

    lax.fori_loop(0, ROUNDS, round_, 0)
    for k in range(NBUF):
        pltpu.make_async_copy(
            bufs.at[k], out.at[pl.ds(w * PER_W, CHUNK)], wsem).wait()


@functools.partial(
    pl.kernel,
    out_type=jax.ShapeDtypeStruct((2, N_PAD, DIM), jnp.float32),
    mesh=_MESH,
    scratch_types=[
        pltpu.VMEM((CHUNKS, CHUNK), jnp.int32),
        pltpu.VMEM((2 * NBUF, CHUNK, DIM), jnp.float32),
        pltpu.VMEM_SHARED((N_PAD, DIM), jnp.float32),
        pltpu.SemaphoreType.DMA,
    ],
    compiler_params=pltpu.CompilerParams(use_tc_tiling_on_sc=False),
)
def _sc_scatter(msg, dst3, zeros32, agg_out, idx_v, bufs, acc, lsem):
    cid = lax.axis_index("c")
    sid = lax.axis_index("s")
    w = sid * 2 + cid
    r0 = sid * STRIPE
    pltpu.sync_copy(zeros32, acc.at[pl.ds(r0, STRIPE)])
    pltpu.sync_copy(dst3.at[w], idx_v)
    for k in range(NBUF):
        pltpu.async_copy(msg.at[pl.ds((w * CHUNKS + k) * CHUNK, CHUNK)],
                         bufs.at[k], lsem)
    plsc.subcore_barrier()

    def round_(i, carry):
        p = lax.rem(i, 2)
        for k in range(NBUF):
            pltpu.make_async_copy(
                msg.at[pl.ds(w * PER_W, CHUNK)], bufs.at[k], lsem).wait()
        @pl.when(i < ROUNDS - 1)
        def _():
            for k in range(NBUF):
                pltpu.async_copy(
                    msg.at[pl.ds((w * CHUNKS + (i + 1) * NBUF + k) * CHUNK,
                                 CHUNK)],
                    bufs.at[(1 - p) * NBUF + k], lsem)
        for k in range(NBUF):
            pltpu.sync_copy(bufs.at[p * NBUF + k],
                            acc.at[idx_v.at[i * NBUF + k]], add=True)
        return carry

    lax.fori_loop(0, ROUNDS, round_, 0)
    plsc.subcore_barrier()
    pltpu.sync_copy(acc.at[pl.ds(r0, STRIPE)],
                    agg_out.at[cid, pl.ds(r0, STRIPE)])


@functools.partial(
    pl.kernel,
    out_type=jax.ShapeDtypeStruct((2, N_PAD, 16), jnp.float32),
    mesh=_MESH,
    scratch_types=[
        pltpu.VMEM((CHUNKS, CHUNK), jnp.int32),
        pltpu.VMEM((CHUNK, 16), jnp.float32),
        pltpu.VMEM_SHARED((N_PAD, 16), jnp.float32),
    ],
    compiler_params=pltpu.CompilerParams(use_tc_tiling_on_sc=False),
)
def _sc_cnt(dst3, zeros16, ones16, cnt_out, idx_v, ones_v, cacc):
    cid = lax.axis_index("c")
    sid = lax.axis_index("s")
    w = sid * 2 + cid
    r0 = sid * STRIPE
    pltpu.sync_copy(zeros16, cacc.at[pl.ds(r0, STRIPE)])
    pltpu.sync_copy(ones16, ones_v)
    pltpu.sync_copy(dst3.at[w], idx_v)
    plsc.subcore_barrier()

    def step(j, carry):
        pltpu.sync_copy(ones_v, cacc.at[idx_v.at[j]], add=True)
        return carry

    lax.fori_loop(0, CHUNKS, step, 0)
    plsc.subcore_barrier()
    pltpu.sync_copy(cacc.at[pl.ds(r0, STRIPE)],
                    cnt_out.at[cid, pl.ds(r0, STRIPE)])


def _lin0_body(nf_ref, w_ref, b_ref, out_ref):
    acc = jnp.dot(nf_ref[...], w_ref[...], preferred_element_type=jnp.float32)
    out_ref[...] = jnp.maximum(acc + b_ref[...], 0.0)


_lin0 = pl.pallas_call(
    _lin0_body,
    grid=(N_PAD // BN,),
    in_specs=[
        pl.BlockSpec((BN, F_IN), lambda i: (i, 0)),
        pl.BlockSpec((F_IN, DIM), lambda i: (0, 0)),
        pl.BlockSpec((1, DIM), lambda i: (0, 0)),
    ],
    out_specs=pl.BlockSpec((BN, DIM), lambda i: (i, 0)),
    out_shape=jax.ShapeDtypeStruct((N_PAD, DIM), jnp.float32),
)


def _msg_body(ef_ref, xg_ref, we1_ref, be1_ref, we2_ref, be2_ref, msg_ref):
    efT = ef_ref[...].T
    h1T = jnp.dot(we1_ref[...], efT, preferred_element_type=jnp.float32)
    h1T = jnp.maximum(h1T + be1_ref[...], 0.0)
    ewT = jnp.dot(we2_ref[...], h1T.astype(jnp.bfloat16),
                  preferred_element_type=jnp.float32)
    xgT = xg_ref[...].T
    ew3 = ewT.reshape(DIM, DIM, BE)
    biasT = jnp.dot(be2_ref[...], xgT,
                    preferred_element_type=jnp.float32)
    msgT = jnp.sum(ew3 * xgT[:, None, :], axis=0) + biasT
    msg_ref[...] = msgT.T


_msg = pl.pallas_call(
    _msg_body,
    grid=(E_PAD // BE,),
    in_specs=[
        pl.BlockSpec((BE, 16), lambda i: (i, 0)),
        pl.BlockSpec((BE, DIM), lambda i: (i, 0)),
        pl.BlockSpec((F_IN, 16), lambda i: (0, 0)),
        pl.BlockSpec((F_IN, 1), lambda i: (0, 0)),
        pl.BlockSpec((DIM * DIM, F_IN), lambda i: (0, 0)),
        pl.BlockSpec((DIM, DIM), lambda i: (0, 0)),
    ],
    out_specs=pl.BlockSpec((BE, DIM), lambda i: (i, 0)),
    out_shape=jax.ShapeDtypeStruct((E_PAD, DIM), jnp.float32),
)


def _gru_body(agg_ref, cnt_ref, h_ref, bconv_ref, wih_ref, whh_ref,
              bih_ref, bhh_ref, out_ref):
    agg = agg_ref[0] + agg_ref[1]
    cnt = jnp.maximum(cnt_ref[0, :, 0:1] + cnt_ref[1, :, 0:1], 1.0)
    m = jnp.maximum(agg / cnt + bconv_ref[...], 0.0)
    h = h_ref[...]
    gi = jnp.dot(m, wih_ref[...], preferred_element_type=jnp.float32)
    gi = gi + bih_ref[...]
    gh = jnp.dot(h, whh_ref[...], preferred_element_type=jnp.float32)
    gh = gh + bhh_ref[...]
    r = jax.nn.sigmoid(gi[:, 0:DIM] + gh[:, 0:DIM])
    z = jax.nn.sigmoid(gi[:, DIM:2 * DIM] + gh[:, DIM:2 * DIM])
    n = jnp.tanh(gi[:, 2 * DIM:] + r * gh[:, 2 * DIM:])
    out_ref[...] = (1.0 - z) * n + z * h


_gru = pl.pallas_call(
    _gru_body,
    grid=(N_PAD // BN,),
    in_specs=[
        pl.BlockSpec((2, BN, DIM), lambda i: (0, i, 0)),
        pl.BlockSpec((2, BN, 16), lambda i: (0, i, 0)),
        pl.BlockSpec((BN, DIM), lambda i: (i, 0)),
        pl.BlockSpec((1, DIM), lambda i: (0, 0)),
        pl.BlockSpec((DIM, 3 * DIM), lambda i: (0, 0)),
        pl.BlockSpec((DIM, 3 * DIM), lambda i: (0, 0)),
        pl.BlockSpec((1, 3 * DIM), lambda i: (0, 0)),
        pl.BlockSpec((1, 3 * DIM), lambda i: (0, 0)),
    ],
    out_specs=pl.BlockSpec((BN, DIM), lambda i: (i, 0)),
    out_shape=jax.ShapeDtypeStruct((N_PAD, DIM), jnp.float32),
)


def _s2s_body(h_ref, gcol_ref, grow_ref, wih_ref, whh_ref, bih_ref, bhh_ref,
              q_ref):
    hfull = h_ref[...]
    gcol = gcol_ref[...]
    grow = grow_ref[...]
    iota_row = lax.broadcasted_iota(jnp.int32, (1, G), 1)
    onehot = (gcol == iota_row).astype(jnp.float32)
    iota_col = lax.broadcasted_iota(jnp.int32, (G, 1), 0)
    onehotT = (iota_col == grow).astype(jnp.float32)
    valid = (gcol < G).astype(jnp.float32)

    q_star = jnp.zeros((G, 2 * DIM), jnp.float32)
    lh = jnp.zeros((G, DIM), jnp.float32)
    lc = jnp.zeros((G, DIM), jnp.float32)
    for _ in range(3):
        gates = (jnp.dot(q_star, wih_ref[...], preferred_element_type=jnp.float32)
                 + bih_ref[...]
                 + jnp.dot(lh, whh_ref[...], preferred_element_type=jnp.float32)
                 + bhh_ref[...])
        i_ = jax.nn.sigmoid(gates[:, 0:DIM])
        f_ = jax.nn.sigmoid(gates[:, DIM:2 * DIM])
        g_ = jnp.tanh(gates[:, 2 * DIM:3 * DIM])
        o_ = jax.nn.sigmoid(gates[:, 3 * DIM:])
        lc = f_ * lc + i_ * g_
        lh = o_ * jnp.tanh(lc)
        q = lh
        qn = jnp.dot(onehot, q, preferred_element_type=jnp.float32)
        e = jnp.sum(hfull * qn, axis=1, keepdims=True)
        e = e * valid
        big = jnp.where(onehot > 0.0, e, -1e30)
        emax = jnp.max(big, axis=0, keepdims=True)
        emax = jnp.where(emax < -1e29, 0.0, emax)
        emax_n = jnp.sum(onehot * emax, axis=1, keepdims=True)
        ee = jnp.exp(e - emax_n) * valid
        denom = jnp.sum(onehot * ee, axis=0, keepdims=True)
        denom_n = jnp.sum(onehot * denom, axis=1, keepdims=True)
        a = ee / (denom_n + 1e-16)
        rvec = jnp.dot(onehotT, a * hfull,
                       preferred_element_type=jnp.float32)
        q_star = jnp.concatenate([q, rvec], axis=1)
    q_ref[...] = q_star


_s2s = pl.pallas_call(
    _s2s_body,
    out_shape=jax.ShapeDtypeStruct((G, 2 * DIM), jnp.float32),
)


def kernel(node_features, edge_features, W0, b0, We1, be1, We2, be2, b_conv,
           gW_ih, gW_hh, gb_ih, gb_hh, lW_ih, lW_hh, lb_ih, lb_hh,
           edge_index, graph_index):
    f32 = jnp.float32
    nf = jnp.zeros((N_PAD, F_IN), f32).at[:N].set(node_features)
    ef = jnp.zeros((E_PAD, 16), f32).at[:E, :11].set(edge_features)
    src3 = (jnp.zeros((E_PAD,), jnp.int32).at[:E].set(edge_index[0])
            .reshape(NW, CHUNKS, CHUNK))
    dst3 = (jnp.full((E_PAD,), N, jnp.int32).at[:E].set(edge_index[1])
            .reshape(NW, CHUNKS, CHUNK))
    gcol = jnp.full((N_PAD, 1), G, jnp.int32).at[:N, 0].set(graph_index)
    grow = gcol.reshape(1, N_PAD)

    W0T = W0.T
    We1p = jnp.zeros((F_IN, 16), f32).at[:, :11].set(We1)
    be1c = be1.reshape(F_IN, 1)
    We2bf = We2.astype(jnp.bfloat16)
    be2m = be2.reshape(DIM, DIM).T
    b0r = b0.reshape(1, DIM)
    bconv = b_conv.reshape(1, DIM)
    gWihT = gW_ih.T
    gWhhT = gW_hh.T
    gbih = gb_ih.reshape(1, 3 * DIM)
    gbhh = gb_hh.reshape(1, 3 * DIM)
    lWihT = lW_ih.T
    lWhhT = lW_hh.T
    lbih = lb_ih.reshape(1, 4 * DIM)
    lbhh = lb_hh.reshape(1, 4 * DIM)

    zeros32 = jnp.zeros((STRIPE, DIM), f32)
    zeros16 = jnp.zeros((STRIPE, 16), f32)
    ones16 = jnp.ones((CHUNK, 16), f32)

    h = _lin0(nf, W0T, b0r)
    cntp = _sc_cnt(dst3, zeros16, ones16)
    for _ in range(3):
        xg = _sc_gather(h, src3)
        msg = _msg(ef, xg, We1p, be1c, We2bf, be2m)
        aggp = _sc_scatter(msg, dst3, zeros32)
        h = _gru(aggp, cntp, h, bconv, gWihT, gWhhT, gbih, gbhh)
    q_star = _s2s(h, gcol, grow, lWihT, lWhhT, lbih, lbhh)
    return (q_star, h[:N])


# speedup vs baseline: 3.6192x; 1.0074x over previous
import functools

import jax
import jax.numpy as jnp
from jax import lax
from jax.experimental import pallas as pl
from jax.experimental.pallas import tpu as pltpu
from jax.experimental.pallas import tpu_sc as plsc

N = 10000
E = 160000
F_IN = 128
DIM = 32
G = 64

NW = 32
CHUNK = 128
CHUNKS = 40
PER_W = CHUNKS * CHUNK
E_PAD = NW * PER_W
N_PAD = 10240
STRIPE = N_PAD // 16
BE = 1024
BN = 1024

_MESH = plsc.VectorSubcoreMesh(
    core_axis_name="c", subcore_axis_name="s", num_cores=2, num_subcores=16)


NBUF = 4
ROUNDS = CHUNKS // NBUF


@functools.partial(
    pl.kernel,
    out_type=jax.ShapeDtypeStruct((E_PAD, DIM), jnp.float32),
    mesh=_MESH,
    scratch_types=[
        pltpu.VMEM((CHUNKS, CHUNK), jnp.int32),
        pltpu.VMEM((NBUF, CHUNK, DIM), jnp.float32),
        pltpu.VMEM_SHARED((N_PAD, DIM), jnp.float32),
        pltpu.SemaphoreType.DMA,
        pltpu.SemaphoreType.DMA,
    ],
    compiler_params=pltpu.CompilerParams(use_tc_tiling_on_sc=False),
)
def _sc_gather(table, src3, out, idx_v, bufs, tbl, gsem, wsem):
    cid = lax.axis_index("c")
    sid = lax.axis_index("s")
    w = sid * 2 + cid
    r0 = sid * STRIPE
    pltpu.sync_copy(table.at[pl.ds(r0, STRIPE)], tbl.at[pl.ds(r0, STRIPE)])
    pltpu.sync_copy(src3.at[w], idx_v)
    plsc.subcore_barrier()

    def round_(i, carry):
        @pl.when(i > 0)
        def _():
            for k in range(NBUF):
                pltpu.make_async_copy(
                    bufs.at[k], out.at[pl.ds(w * PER_W, CHUNK)], wsem).wait()
        gds = [
            pltpu.async_copy(tbl.at[idx_v.at[i * NBUF + k]], bufs.at[k], gsem)
            for k in range(NBUF)
        ]
        for k in range(NBUF):
            gds[k].wait()
        for k in range(NBUF):
            pltpu.async_copy(
                bufs.at[k],
                out.at[pl.ds(w * PER_W + (i * NBUF + k) * CHUNK, CHUNK)],
                wsem)
        return carry

    lax.fori_loop(0, ROUNDS, round_, 0)
    for k in range(NBUF):
        pltpu.make_async_copy(
            bufs.at[k], out.at[pl.ds(w * PER_W, CHUNK)], wsem).wait()


@functools.partial(
    pl.kernel,
    out_type=jax.ShapeDtypeStruct((2, N_PAD, DIM), jnp.float32),
    mesh=_MESH,
    scratch_types=[
        pltpu.VMEM((CHUNKS, CHUNK), jnp.int32),
        pltpu.VMEM((2 * NBUF, CHUNK, DIM), jnp.float32),
        pltpu.VMEM_SHARED((N_PAD, DIM), jnp.float32),
        pltpu.SemaphoreType.DMA,
    ],
    compiler_params=pltpu.CompilerParams(use_tc_tiling_on_sc=False),
)
def _sc_scatter(msg, dst3, zeros32, agg_out, idx_v, bufs, acc, lsem):
    cid = lax.axis_index("c")
    sid = lax.axis_index("s")
    w = sid * 2 + cid
    r0 = sid * STRIPE
    pltpu.sync_copy(zeros32, acc.at[pl.ds(r0, STRIPE)])
    pltpu.sync_copy(dst3.at[w], idx_v)
    for k in range(NBUF):
        pltpu.async_copy(msg.at[pl.ds((w * CHUNKS + k) * CHUNK, CHUNK)],
                         bufs.at[k], lsem)
    plsc.subcore_barrier()

    def round_(i, carry):
        p = lax.rem(i, 2)
        for k in range(NBUF):
            pltpu.make_async_copy(
                msg.at[pl.ds(w * PER_W, CHUNK)], bufs.at[k], lsem).wait()
        @pl.when(i < ROUNDS - 1)
        def _():
            for k in range(NBUF):
                pltpu.async_copy(
                    msg.at[pl.ds((w * CHUNKS + (i + 1) * NBUF + k) * CHUNK,
                                 CHUNK)],
                    bufs.at[(1 - p) * NBUF + k], lsem)
        for k in range(NBUF):
            pltpu.sync_copy(bufs.at[p * NBUF + k],
                            acc.at[idx_v.at[i * NBUF + k]], add=True)
        return carry

    lax.fori_loop(0, ROUNDS, round_, 0)
    plsc.subcore_barrier()
    pltpu.sync_copy(acc.at[pl.ds(r0, STRIPE)],
                    agg_out.at[cid, pl.ds(r0, STRIPE)])


@functools.partial(
    pl.kernel,
    out_type=jax.ShapeDtypeStruct((2, N_PAD, 16), jnp.float32),
    mesh=_MESH,
    scratch_types=[
        pltpu.VMEM((CHUNKS, CHUNK), jnp.int32),
        pltpu.VMEM((CHUNK, 16), jnp.float32),
        pltpu.VMEM_SHARED((N_PAD, 16), jnp.float32),
    ],
    compiler_params=pltpu.CompilerParams(use_tc_tiling_on_sc=False),
)
def _sc_cnt(dst3, zeros16, ones16, cnt_out, idx_v, ones_v, cacc):
    cid = lax.axis_index("c")
    sid = lax.axis_index("s")
    w = sid * 2 + cid
    r0 = sid * STRIPE
    pltpu.sync_copy(zeros16, cacc.at[pl.ds(r0, STRIPE)])
    pltpu.sync_copy(ones16, ones_v)
    pltpu.sync_copy(dst3.at[w], idx_v)
    plsc.subcore_barrier()

    def step(j, carry):
        pltpu.sync_copy(ones_v, cacc.at[idx_v.at[j]], add=True)
        return carry

    lax.fori_loop(0, CHUNKS, step, 0)
    plsc.subcore_barrier()
    pltpu.sync_copy(cacc.at[pl.ds(r0, STRIPE)],
                    cnt_out.at[cid, pl.ds(r0, STRIPE)])


def _lin0_body(nf_ref, w_ref, b_ref, out_ref):
    acc = jnp.dot(nf_ref[...], w_ref[...], preferred_element_type=jnp.float32)
    out_ref[...] = jnp.maximum(acc + b_ref[...], 0.0)


_lin0 = pl.pallas_call(
    _lin0_body,
    grid=(N_PAD // BN,),
    in_specs=[
        pl.BlockSpec((BN, F_IN), lambda i: (i, 0)),
        pl.BlockSpec((F_IN, DIM), lambda i: (0, 0)),
        pl.BlockSpec((1, DIM), lambda i: (0, 0)),
    ],
    out_specs=pl.BlockSpec((BN, DIM), lambda i: (i, 0)),
    out_shape=jax.ShapeDtypeStruct((N_PAD, DIM), jnp.float32),
)


def _msg_body(efT_ref, xg_ref, we1_ref, be1_ref, we2_ref, be2_ref, msg_ref):
    h1T = jnp.dot(we1_ref[...], efT_ref[...],
                  preferred_element_type=jnp.float32)
    h1T = jnp.maximum(h1T + be1_ref[...], 0.0)
    ewT = jnp.dot(we2_ref[...], h1T.astype(jnp.bfloat16),
                  preferred_element_type=jnp.float32)
    xgT = xg_ref[...].T
    acc = jnp.dot(be2_ref[...], xgT,
                  preferred_element_type=jnp.float32)
    for i in range(DIM):
        acc = acc + ewT[DIM * i:DIM * (i + 1), :] * xgT[i:i + 1, :]
    msg_ref[...] = acc.T


_msg = pl.pallas_call(
    _msg_body,
    grid=(E_PAD // BE,),
    in_specs=[
        pl.BlockSpec((16, BE), lambda i: (0, i)),
        pl.BlockSpec((BE, DIM), lambda i: (i, 0)),
        pl.BlockSpec((F_IN, 16), lambda i: (0, 0)),
        pl.BlockSpec((F_IN, 1), lambda i: (0, 0)),
        pl.BlockSpec((DIM * DIM, F_IN), lambda i: (0, 0)),
        pl.BlockSpec((DIM, DIM), lambda i: (0, 0)),
    ],
    out_specs=pl.BlockSpec((BE, DIM), lambda i: (i, 0)),
    out_shape=jax.ShapeDtypeStruct((E_PAD, DIM), jnp.float32),
)


def _gru_body(agg_ref, cnt_ref, h_ref, bconv_ref, wih_ref, whh_ref,
              bih_ref, bhh_ref, out_ref):
    agg = agg_ref[0] + agg_ref[1]
    cnt = jnp.maximum(cnt_ref[0, :, 0:1] + cnt_ref[1, :, 0:1], 1.0)
    m = jnp.maximum(agg / cnt + bconv_ref[...], 0.0)
    h = h_ref[...]
    gi = jnp.dot(m, wih_ref[...], preferred_element_type=jnp.float32)
    gi = gi + bih_ref[...]
    gh = jnp.dot(h, whh_ref[...], preferred_element_type=jnp.float32)
    gh = gh + bhh_ref[...]
    r = jax.nn.sigmoid(gi[:, 0:DIM] + gh[:, 0:DIM])
    z = jax.nn.sigmoid(gi[:, DIM:2 * DIM] + gh[:, DIM:2 * DIM])
    n = jnp.tanh(gi[:, 2 * DIM:] + r * gh[:, 2 * DIM:])
    out_ref[...] = (1.0 - z) * n + z * h


_gru = pl.pallas_call(
    _gru_body,
    grid=(N_PAD // BN,),
    in_specs=[
        pl.BlockSpec((2, BN, DIM), lambda i: (0, i, 0)),
        pl.BlockSpec((2, BN, 16), lambda i: (0, i, 0)),
        pl.BlockSpec((BN, DIM), lambda i: (i, 0)),
        pl.BlockSpec((1, DIM), lambda i: (0, 0)),
        pl.BlockSpec((DIM, 3 * DIM), lambda i: (0, 0)),
        pl.BlockSpec((DIM, 3 * DIM), lambda i: (0, 0)),
        pl.BlockSpec((1, 3 * DIM), lambda i: (0, 0)),
        pl.BlockSpec((1, 3 * DIM), lambda i: (0, 0)),
    ],
    out_specs=pl.BlockSpec((BN, DIM), lambda i: (i, 0)),
    out_shape=jax.ShapeDtypeStruct((N_PAD, DIM), jnp.float32),
)


def _s2s_body(h_ref, gcol_ref, grow_ref, wih_ref, whh_ref, bih_ref, bhh_ref,
              q_ref):
    hfull = h_ref[...]
    gcol = gcol_ref[...]
    grow = grow_ref[...]
    iota_row = lax.broadcasted_iota(jnp.int32, (1, G), 1)
    onehot = (gcol == iota_row).astype(jnp.float32)
    iota_col = lax.broadcasted_iota(jnp.int32, (G, 1), 0)
    onehotT = (iota_col == grow).astype(jnp.float32)
    valid = (gcol < G).astype(jnp.float32)

    q_star = jnp.zeros((G, 2 * DIM), jnp.float32)
    lh = jnp.zeros((G, DIM), jnp.float32)
    lc = jnp.zeros((G, DIM), jnp.float32)
    for _ in range(3):
        gates = (jnp.dot(q_star, wih_ref[...], preferred_element_type=jnp.float32)
                 + bih_ref[...]
                 + jnp.dot(lh, whh_ref[...], preferred_element_type=jnp.float32)
                 + bhh_ref[...])
        i_ = jax.nn.sigmoid(gates[:, 0:DIM])
        f_ = jax.nn.sigmoid(gates[:, DIM:2 * DIM])
        g_ = jnp.tanh(gates[:, 2 * DIM:3 * DIM])
        o_ = jax.nn.sigmoid(gates[:, 3 * DIM:])
        lc = f_ * lc + i_ * g_
        lh = o_ * jnp.tanh(lc)
        q = lh
        qn = jnp.dot(onehot, q, preferred_element_type=jnp.float32)
        e = jnp.sum(hfull * qn, axis=1, keepdims=True)
        e = e * valid
        big = jnp.where(onehot > 0.0, e, -1e30)
        emax = jnp.max(big, axis=0, keepdims=True)
        emax = jnp.where(emax < -1e29, 0.0, emax)
        emax_n = jnp.sum(onehot * emax, axis=1, keepdims=True)
        ee = jnp.exp(e - emax_n) * valid
        denom = jnp.sum(onehot * ee, axis=0, keepdims=True)
        denom_n = jnp.sum(onehot * denom, axis=1, keepdims=True)
        a = ee / (denom_n + 1e-16)
        rvec = jnp.dot(onehotT, a * hfull,
                       preferred_element_type=jnp.float32)
        q_star = jnp.concatenate([q, rvec], axis=1)
    q_ref[...] = q_star


_s2s = pl.pallas_call(
    _s2s_body,
    out_shape=jax.ShapeDtypeStruct((G, 2 * DIM), jnp.float32),
)


def kernel(node_features, edge_features, W0, b0, We1, be1, We2, be2, b_conv,
           gW_ih, gW_hh, gb_ih, gb_hh, lW_ih, lW_hh, lb_ih, lb_hh,
           edge_index, graph_index):
    f32 = jnp.float32
    nf = jnp.zeros((N_PAD, F_IN), f32).at[:N].set(node_features)
    efT = jnp.zeros((16, E_PAD), f32).at[:11, :E].set(edge_features.T)
    src3 = (jnp.zeros((E_PAD,), jnp.int32).at[:E].set(edge_index[0])
            .reshape(NW, CHUNKS, CHUNK))
    dst3 = (jnp.full((E_PAD,), N, jnp.int32).at[:E].set(edge_index[1])
            .reshape(NW, CHUNKS, CHUNK))
    gcol = jnp.full((N_PAD, 1), G, jnp.int32).at[:N, 0].set(graph_index)
    grow = gcol.reshape(1, N_PAD)

    W0T = W0.T
    We1p = jnp.zeros((F_IN, 16), f32).at[:, :11].set(We1)
    be1c = be1.reshape(F_IN, 1)
    We2bf = We2.astype(jnp.bfloat16)
    be2m = be2.reshape(DIM, DIM).T
    b0r = b0.reshape(1, DIM)
    bconv = b_conv.reshape(1, DIM)
    gWihT = gW_ih.T
    gWhhT = gW_hh.T
    gbih = gb_ih.reshape(1, 3 * DIM)
    gbhh = gb_hh.reshape(1, 3 * DIM)
    lWihT = lW_ih.T
    lWhhT = lW_hh.T
    lbih = lb_ih.reshape(1, 4 * DIM)
    lbhh = lb_hh.reshape(1, 4 * DIM)

    zeros32 = jnp.zeros((STRIPE, DIM), f32)
    zeros16 = jnp.zeros((STRIPE, 16), f32)
    ones16 = jnp.ones((CHUNK, 16), f32)

    h = _lin0(nf, W0T, b0r)
    cntp = _sc_cnt(dst3, zeros16, ones16)
    for _ in range(3):
        xg = _sc_gather(h, src3)
        msg = _msg(efT, xg, We1p, be1c, We2bf, be2m)
        aggp = _sc_scatter(msg, dst3, zeros32)
        h = _gru(aggp, cntp, h, bconv, gWihT, gWhhT, gbih, gbhh)
    q_star = _s2s(h, gcol, grow, lWihT, lWhhT, lbih, lbhh)
    return (q_star, h[:N])


# speedup vs baseline: 3.8473x; 1.0630x over previous
import functools

import jax
import jax.numpy as jnp
from jax import lax
from jax.experimental import pallas as pl
from jax.experimental.pallas import tpu as pltpu
from jax.experimental.pallas import tpu_sc as plsc

N = 10000
E = 160000
F_IN = 128
DIM = 32
G = 64

NW = 32
CHUNK = 128
CHUNKS = 40
PER_W = CHUNKS * CHUNK
E_PAD = NW * PER_W
N_PAD = 10240
STRIPE = N_PAD // 16
BE = 2048
BN = 1024

_MESH = plsc.VectorSubcoreMesh(
    core_axis_name="c", subcore_axis_name="s", num_cores=2, num_subcores=16)


NBUF = 4
ROUNDS = CHUNKS // NBUF


@functools.partial(
    pl.kernel,
    out_type=jax.ShapeDtypeStruct((E_PAD, DIM), jnp.float32),
    mesh=_MESH,
    scratch_types=[
        pltpu.VMEM((CHUNKS, CHUNK), jnp.int32),
        pltpu.VMEM((NBUF, CHUNK, DIM), jnp.float32),
        pltpu.VMEM_SHARED((N_PAD, DIM), jnp.float32),
        pltpu.SemaphoreType.DMA,
        pltpu.SemaphoreType.DMA,
    ],
    compiler_params=pltpu.CompilerParams(use_tc_tiling_on_sc=False),
)
def _sc_gather(table, src3, out, idx_v, bufs, tbl, gsem, wsem):
    cid = lax.axis_index("c")
    sid = lax.axis_index("s")
    w = sid * 2 + cid
    r0 = sid * STRIPE
    pltpu.sync_copy(table.at[pl.ds(r0, STRIPE)], tbl.at[pl.ds(r0, STRIPE)])
    pltpu.sync_copy(src3.at[w], idx_v)
    plsc.subcore_barrier()

    def round_(i, carry):
        @pl.when(i > 0)
        def _():
            for k in range(NBUF):
                pltpu.make_async_copy(
                    bufs.at[k], out.at[pl.ds(w * PER_W, CHUNK)], wsem).wait()
        gds = [
            pltpu.async_copy(tbl.at[idx_v.at[i * NBUF + k]], bufs.at[k], gsem)
            for k in range(NBUF)
        ]
        for k in range(NBUF):
            gds[k].wait()
        for k in range(NBUF):
            pltpu.async_copy(
                bufs.at[k],
                out.at[pl.ds(w * PER_W + (i * NBUF + k) * CHUNK, CHUNK)],
                wsem)
        return carry

    lax.fori_loop(0, ROUNDS, round_, 0)
    for k in range(NBUF):
        pltpu.make_async_copy(
            bufs.at[k], out.at[pl.ds(w * PER_W, CHUNK)], wsem).wait()


@functools.partial(
    pl.kernel,
    out_type=jax.ShapeDtypeStruct((2, N_PAD, DIM), jnp.float32),
    mesh=_MESH,
    scratch_types=[
        pltpu.VMEM((CHUNKS, CHUNK), jnp.int32),
        pltpu.VMEM((2 * NBUF, CHUNK, DIM), jnp.float32),
        pltpu.VMEM_SHARED((N_PAD, DIM), jnp.float32),
        pltpu.SemaphoreType.DMA,
    ],
    compiler_params=pltpu.CompilerParams(use_tc_tiling_on_sc=False),
)
def _sc_scatter(msg, dst3, zeros32, agg_out, idx_v, bufs, acc, lsem):
    cid = lax.axis_index("c")
    sid = lax.axis_index("s")
    w = sid * 2 + cid
    r0 = sid * STRIPE
    pltpu.sync_copy(zeros32, acc.at[pl.ds(r0, STRIPE)])
    pltpu.sync_copy(dst3.at[w], idx_v)
    for k in range(NBUF):
        pltpu.async_copy(msg.at[pl.ds((w * CHUNKS + k) * CHUNK, CHUNK)],
                         bufs.at[k], lsem)
    plsc.subcore_barrier()

    def round_(i, carry):
        p = lax.rem(i, 2)
        for k in range(NBUF):
            pltpu.make_async_copy(
                msg.at[pl.ds(w * PER_W, CHUNK)], bufs.at[k], lsem).wait()
        @pl.when(i < ROUNDS - 1)
        def _():
            for k in range(NBUF):
                pltpu.async_copy(
                    msg.at[pl.ds((w * CHUNKS + (i + 1) * NBUF + k) * CHUNK,
                                 CHUNK)],
                    bufs.at[(1 - p) * NBUF + k], lsem)
        for k in range(NBUF):
            pltpu.sync_copy(bufs.at[p * NBUF + k],
                            acc.at[idx_v.at[i * NBUF + k]], add=True)
        return carry

    lax.fori_loop(0, ROUNDS, round_, 0)
    plsc.subcore_barrier()
    pltpu.sync_copy(acc.at[pl.ds(r0, STRIPE)],
                    agg_out.at[cid, pl.ds(r0, STRIPE)])


@functools.partial(
    pl.kernel,
    out_type=jax.ShapeDtypeStruct((2, N_PAD, 16), jnp.float32),
    mesh=_MESH,
    scratch_types=[
        pltpu.VMEM((CHUNKS, CHUNK), jnp.int32),
        pltpu.VMEM((CHUNK, 16), jnp.float32),
        pltpu.VMEM_SHARED((N_PAD, 16), jnp.float32),
    ],
    compiler_params=pltpu.CompilerParams(use_tc_tiling_on_sc=False),
)
def _sc_cnt(dst3, zeros16, ones16, cnt_out, idx_v, ones_v, cacc):
    cid = lax.axis_index("c")
    sid = lax.axis_index("s")
    w = sid * 2 + cid
    r0 = sid * STRIPE
    pltpu.sync_copy(zeros16, cacc.at[pl.ds(r0, STRIPE)])
    pltpu.sync_copy(ones16, ones_v)
    pltpu.sync_copy(dst3.at[w], idx_v)
    plsc.subcore_barrier()

    def step(j, carry):
        pltpu.sync_copy(ones_v, cacc.at[idx_v.at[j]], add=True)
        return carry

    lax.fori_loop(0, CHUNKS, step, 0)
    plsc.subcore_barrier()
    pltpu.sync_copy(cacc.at[pl.ds(r0, STRIPE)],
                    cnt_out.at[cid, pl.ds(r0, STRIPE)])


def _lin0_body(nf_ref, w_ref, b_ref, out_ref):
    acc = jnp.dot(nf_ref[...], w_ref[...], preferred_element_type=jnp.float32)
    out_ref[...] = jnp.maximum(acc + b_ref[...], 0.0)


_lin0 = pl.pallas_call(
    _lin0_body,
    grid=(N_PAD // BN,),
    in_specs=[
        pl.BlockSpec((BN, F_IN), lambda i: (i, 0)),
        pl.BlockSpec((F_IN, DIM), lambda i: (0, 0)),
        pl.BlockSpec((1, DIM), lambda i: (0, 0)),
    ],
    out_specs=pl.BlockSpec((BN, DIM), lambda i: (i, 0)),
    out_shape=jax.ShapeDtypeStruct((N_PAD, DIM), jnp.float32),
)


def _msg_body(efT_ref, xg_ref, we1_ref, be1_ref, we2_ref, be2_ref, msg_ref):
    h1T = jnp.dot(we1_ref[...], efT_ref[...],
                  preferred_element_type=jnp.float32)
    h1T = jnp.maximum(h1T + be1_ref[...], 0.0)
    ewT = jnp.dot(we2_ref[...], h1T.astype(jnp.bfloat16),
                  preferred_element_type=jnp.float32)
    xgT = xg_ref[...].T
    bias = jnp.dot(be2_ref[...], xgT,
                   preferred_element_type=jnp.float32)
    cols = []
    for b in range(BE // 128):
        lo, hi = b * 128, (b + 1) * 128
        acc = bias[:, lo:hi]
        for i in range(DIM):
            acc = acc + (ewT[DIM * i:DIM * (i + 1), lo:hi]
                         * xgT[i:i + 1, lo:hi])
        cols.append(acc)
    msg_ref[...] = jnp.concatenate(cols, axis=1).T


_msg = pl.pallas_call(
    _msg_body,
    grid=(E_PAD // BE,),
    in_specs=[
        pl.BlockSpec((16, BE), lambda i: (0, i)),
        pl.BlockSpec((BE, DIM), lambda i: (i, 0)),
        pl.BlockSpec((F_IN, 16), lambda i: (0, 0)),
        pl.BlockSpec((F_IN, 1), lambda i: (0, 0)),
        pl.BlockSpec((DIM * DIM, F_IN), lambda i: (0, 0)),
        pl.BlockSpec((DIM, DIM), lambda i: (0, 0)),
    ],
    out_specs=pl.BlockSpec((BE, DIM), lambda i: (i, 0)),
    out_shape=jax.ShapeDtypeStruct((E_PAD, DIM), jnp.float32),
)


def _gru_body(agg_ref, cnt_ref, h_ref, bconv_ref, wih_ref, whh_ref,
              bih_ref, bhh_ref, out_ref):
    agg = agg_ref[0] + agg_ref[1]
    cnt = jnp.maximum(cnt_ref[0, :, 0:1] + cnt_ref[1, :, 0:1], 1.0)
    m = jnp.maximum(agg / cnt + bconv_ref[...], 0.0)
    h = h_ref[...]
    gi = jnp.dot(m, wih_ref[...], preferred_element_type=jnp.float32)
    gi = gi + bih_ref[...]
    gh = jnp.dot(h, whh_ref[...], preferred_element_type=jnp.float32)
    gh = gh + bhh_ref[...]
    r = jax.nn.sigmoid(gi[:, 0:DIM] + gh[:, 0:DIM])
    z = jax.nn.sigmoid(gi[:, DIM:2 * DIM] + gh[:, DIM:2 * DIM])
    n = jnp.tanh(gi[:, 2 * DIM:] + r * gh[:, 2 * DIM:])
    out_ref[...] = (1.0 - z) * n + z * h


_gru = pl.pallas_call(
    _gru_body,
    grid=(N_PAD // BN,),
    in_specs=[
        pl.BlockSpec((2, BN, DIM), lambda i: (0, i, 0)),
        pl.BlockSpec((2, BN, 16), lambda i: (0, i, 0)),
        pl.BlockSpec((BN, DIM), lambda i: (i, 0)),
        pl.BlockSpec((1, DIM), lambda i: (0, 0)),
        pl.BlockSpec((DIM, 3 * DIM), lambda i: (0, 0)),
        pl.BlockSpec((DIM, 3 * DIM), lambda i: (0, 0)),
        pl.BlockSpec((1, 3 * DIM), lambda i: (0, 0)),
        pl.BlockSpec((1, 3 * DIM), lambda i: (0, 0)),
    ],
    out_specs=pl.BlockSpec((BN, DIM), lambda i: (i, 0)),
    out_shape=jax.ShapeDtypeStruct((N_PAD, DIM), jnp.float32),
)


def _s2s_body(h_ref, gcol_ref, grow_ref, wih_ref, whh_ref, bih_ref, bhh_ref,
              q_ref):
    hfull = h_ref[...]
    gcol = gcol_ref[...]
    grow = grow_ref[...]
    iota_row = lax.broadcasted_iota(jnp.int32, (1, G), 1)
    onehot = (gcol == iota_row).astype(jnp.float32)
    iota_col = lax.broadcasted_iota(jnp.int32, (G, 1), 0)
    onehotT = (iota_col == grow).astype(jnp.float32)
    valid = (gcol < G).astype(jnp.float32)

    q_star = jnp.zeros((G, 2 * DIM), jnp.float32)
    lh = jnp.zeros((G, DIM), jnp.float32)
    lc = jnp.zeros((G, DIM), jnp.float32)
    for _ in range(3):
        gates = (jnp.dot(q_star, wih_ref[...], preferred_element_type=jnp.float32)
                 + bih_ref[...]
                 + jnp.dot(lh, whh_ref[...], preferred_element_type=jnp.float32)
                 + bhh_ref[...])
        i_ = jax.nn.sigmoid(gates[:, 0:DIM])
        f_ = jax.nn.sigmoid(gates[:, DIM:2 * DIM])
        g_ = jnp.tanh(gates[:, 2 * DIM:3 * DIM])
        o_ = jax.nn.sigmoid(gates[:, 3 * DIM:])
        lc = f_ * lc + i_ * g_
        lh = o_ * jnp.tanh(lc)
        q = lh
        qn = jnp.dot(onehot, q, preferred_element_type=jnp.float32)
        e = jnp.sum(hfull * qn, axis=1, keepdims=True)
        e = e * valid
        big = jnp.where(onehot > 0.0, e, -1e30)
        emax = jnp.max(big, axis=0, keepdims=True)
        emax = jnp.where(emax < -1e29, 0.0, emax)
        emax_n = jnp.sum(onehot * emax, axis=1, keepdims=True)
        ee = jnp.exp(e - emax_n) * valid
        denom = jnp.sum(onehot * ee, axis=0, keepdims=True)
        denom_n = jnp.sum(onehot * denom, axis=1, keepdims=True)
        a = ee / (denom_n + 1e-16)
        rvec = jnp.dot(onehotT, a * hfull,
                       preferred_element_type=jnp.float32)
        q_star = jnp.concatenate([q, rvec], axis=1)
    q_ref[...] = q_star


_s2s = pl.pallas_call(
    _s2s_body,
    out_shape=jax.ShapeDtypeStruct((G, 2 * DIM), jnp.float32),
)


def kernel(node_features, edge_features, W0, b0, We1, be1, We2, be2, b_conv,
           gW_ih, gW_hh, gb_ih, gb_hh, lW_ih, lW_hh, lb_ih, lb_hh,
           edge_index, graph_index):
    f32 = jnp.float32
    nf = jnp.zeros((N_PAD, F_IN), f32).at[:N].set(node_features)
    efT = jnp.zeros((16, E_PAD), f32).at[:11, :E].set(edge_features.T)
    src3 = (jnp.zeros((E_PAD,), jnp.int32).at[:E].set(edge_index[0])
            .reshape(NW, CHUNKS, CHUNK))
    dst3 = (jnp.full((E_PAD,), N, jnp.int32).at[:E].set(edge_index[1])
            .reshape(NW, CHUNKS, CHUNK))
    gcol = jnp.full((N_PAD, 1), G, jnp.int32).at[:N, 0].set(graph_index)
    grow = gcol.reshape(1, N_PAD)

    W0T = W0.T
    We1p = jnp.zeros((F_IN, 16), f32).at[:, :11].set(We1)
    be1c = be1.reshape(F_IN, 1)
    We2bf = We2.astype(jnp.bfloat16)
    be2m = be2.reshape(DIM, DIM).T
    b0r = b0.reshape(1, DIM)
    bconv = b_conv.reshape(1, DIM)
    gWihT = gW_ih.T
    gWhhT = gW_hh.T
    gbih = gb_ih.reshape(1, 3 * DIM)
    gbhh = gb_hh.reshape(1, 3 * DIM)
    lWihT = lW_ih.T
    lWhhT = lW_hh.T
    lbih = lb_ih.reshape(1, 4 * DIM)
    lbhh = lb_hh.reshape(1, 4 * DIM)

    zeros32 = jnp.zeros((STRIPE, DIM), f32)
    zeros16 = jnp.zeros((STRIPE, 16), f32)
    ones16 = jnp.ones((CHUNK, 16), f32)

    h = _lin0(nf, W0T, b0r)
    cntp = _sc_cnt(dst3, zeros16, ones16)
    for _ in range(3):
        xg = _sc_gather(h, src3)
        msg = _msg(efT, xg, We1p, be1c, We2bf, be2m)
        aggp = _sc_scatter(msg, dst3, zeros32)
        h = _gru(aggp, cntp, h, bconv, gWihT, gWhhT, gbih, gbhh)
    q_star = _s2s(h, gcol, grow, lWihT, lWhhT, lbih, lbhh)
    return (q_star, h[:N])


# speedup vs baseline: 4.0015x; 1.0401x over previous
import functools

import jax
import jax.numpy as jnp
from jax import lax
from jax.experimental import pallas as pl
from jax.experimental.pallas import tpu as pltpu
from jax.experimental.pallas import tpu_sc as plsc

N = 10000
E = 160000
F_IN = 128
DIM = 32
G = 64

NW = 32
CHUNK = 128
CHUNKS = 40
PER_W = CHUNKS * CHUNK
E_PAD = NW * PER_W
N_PAD = 10240
STRIPE = N_PAD // 16
BE = 4096
BN = 1024

_MESH = plsc.VectorSubcoreMesh(
    core_axis_name="c", subcore_axis_name="s", num_cores=2, num_subcores=16)


NBUF = 4
ROUNDS = CHUNKS // NBUF


@functools.partial(
    pl.kernel,
    out_type=jax.ShapeDtypeStruct((E_PAD, DIM), jnp.float32),
    mesh=_MESH,
    scratch_types=[
        pltpu.VMEM((CHUNKS, CHUNK), jnp.int32),
        pltpu.VMEM((NBUF, CHUNK, DIM), jnp.float32),
        pltpu.VMEM_SHARED((N_PAD, DIM), jnp.float32),
        pltpu.SemaphoreType.DMA,
        pltpu.SemaphoreType.DMA,
    ],
    compiler_params=pltpu.CompilerParams(use_tc_tiling_on_sc=False),
)
def _sc_gather(table, src3, out, idx_v, bufs, tbl, gsem, wsem):
    cid = lax.axis_index("c")
    sid = lax.axis_index("s")
    w = sid * 2 + cid
    r0 = sid * STRIPE
    pltpu.sync_copy(table.at[pl.ds(r0, STRIPE)], tbl.at[pl.ds(r0, STRIPE)])
    pltpu.sync_copy(src3.at[w], idx_v)
    plsc.subcore_barrier()

    def round_(i, carry):
        @pl.when(i > 0)
        def _():
            for k in range(NBUF):
                pltpu.make_async_copy(
                    bufs.at[k], out.at[pl.ds(w * PER_W, CHUNK)], wsem).wait()
        gds = [
            pltpu.async_copy(tbl.at[idx_v.at[i * NBUF + k]], bufs.at[k], gsem)
            for k in range(NBUF)
        ]
        for k in range(NBUF):
            gds[k].wait()
        for k in range(NBUF):
            pltpu.async_copy(
                bufs.at[k],
                out.at[pl.ds(w * PER_W + (i * NBUF + k) * CHUNK, CHUNK)],
                wsem)
        return carry

    lax.fori_loop(0, ROUNDS, round_, 0)
    for k in range(NBUF):
        pltpu.make_async_copy(
            bufs.at[k], out.at[pl.ds(w * PER_W, CHUNK)], wsem).wait()


@functools.partial(
    pl.kernel,
    out_type=jax.ShapeDtypeStruct((2, N_PAD, DIM), jnp.float32),
    mesh=_MESH,
    scratch_types=[
        pltpu.VMEM((CHUNKS, CHUNK), jnp.int32),
        pltpu.VMEM((2 * NBUF, CHUNK, DIM), jnp.float32),
        pltpu.VMEM_SHARED((N_PAD, DIM), jnp.float32),
        pltpu.SemaphoreType.DMA,
    ],
    compiler_params=pltpu.CompilerParams(use_tc_tiling_on_sc=False),
)
def _sc_scatter(msg, dst3, zeros32, agg_out, idx_v, bufs, acc, lsem):
    cid = lax.axis_index("c")
    sid = lax.axis_index("s")
    w = sid * 2 + cid
    r0 = sid * STRIPE
    pltpu.sync_copy(zeros32, acc.at[pl.ds(r0, STRIPE)])
    pltpu.sync_copy(dst3.at[w], idx_v)
    for k in range(NBUF):
        pltpu.async_copy(msg.at[pl.ds((w * CHUNKS + k) * CHUNK, CHUNK)],
                         bufs.at[k], lsem)
    plsc.subcore_barrier()

    def round_(i, carry):
        p = lax.rem(i, 2)
        for k in range(NBUF):
            pltpu.make_async_copy(
                msg.at[pl.ds(w * PER_W, CHUNK)], bufs.at[k], lsem).wait()
        @pl.when(i < ROUNDS - 1)
        def _():
            for k in range(NBUF):
                pltpu.async_copy(
                    msg.at[pl.ds((w * CHUNKS + (i + 1) * NBUF + k) * CHUNK,
                                 CHUNK)],
                    bufs.at[(1 - p) * NBUF + k], lsem)
        for k in range(NBUF):
            pltpu.sync_copy(bufs.at[p * NBUF + k],
                            acc.at[idx_v.at[i * NBUF + k]], add=True)
        return carry

    lax.fori_loop(0, ROUNDS, round_, 0)
    plsc.subcore_barrier()
    pltpu.sync_copy(acc.at[pl.ds(r0, STRIPE)],
                    agg_out.at[cid, pl.ds(r0, STRIPE)])


@functools.partial(
    pl.kernel,
    out_type=jax.ShapeDtypeStruct((2, N_PAD, 16), jnp.float32),
    mesh=_MESH,
    scratch_types=[
        pltpu.VMEM((CHUNKS, CHUNK), jnp.int32),
        pltpu.VMEM((CHUNK, 16), jnp.float32),
        pltpu.VMEM_SHARED((N_PAD, 16), jnp.float32),
    ],
    compiler_params=pltpu.CompilerParams(use_tc_tiling_on_sc=False),
)
def _sc_cnt(dst3, zeros16, ones16, cnt_out, idx_v, ones_v, cacc):
    cid = lax.axis_index("c")
    sid = lax.axis_index("s")
    w = sid * 2 + cid
    r0 = sid * STRIPE
    pltpu.sync_copy(zeros16, cacc.at[pl.ds(r0, STRIPE)])
    pltpu.sync_copy(ones16, ones_v)
    pltpu.sync_copy(dst3.at[w], idx_v)
    plsc.subcore_barrier()

    def step(j, carry):
        pltpu.sync_copy(ones_v, cacc.at[idx_v.at[j]], add=True)
        return carry

    lax.fori_loop(0, CHUNKS, step, 0)
    plsc.subcore_barrier()
    pltpu.sync_copy(cacc.at[pl.ds(r0, STRIPE)],
                    cnt_out.at[cid, pl.ds(r0, STRIPE)])


def _lin0_body(nf_ref, w_ref, b_ref, out_ref):
    acc = jnp.dot(nf_ref[...], w_ref[...], preferred_element_type=jnp.float32)
    out_ref[...] = jnp.maximum(acc + b_ref[...], 0.0)


_lin0 = pl.pallas_call(
    _lin0_body,
    grid=(N_PAD // BN,),
    in_specs=[
        pl.BlockSpec((BN, F_IN), lambda i: (i, 0)),
        pl.BlockSpec((F_IN, DIM), lambda i: (0, 0)),
        pl.BlockSpec((1, DIM), lambda i: (0, 0)),
    ],
    out_specs=pl.BlockSpec((BN, DIM), lambda i: (i, 0)),
    out_shape=jax.ShapeDtypeStruct((N_PAD, DIM), jnp.float32),
)


def _msg_body(efT_ref, xg_ref, we1_ref, be1_ref, we2_ref, be2_ref, msg_ref):
    h1T = jnp.dot(we1_ref[...], efT_ref[...],
                  preferred_element_type=jnp.float32)
    h1T = jnp.maximum(h1T + be1_ref[...], 0.0)
    ewT = jnp.dot(we2_ref[...], h1T.astype(jnp.bfloat16),
                  preferred_element_type=jnp.float32)
    xgT = xg_ref[...].T
    bias = jnp.dot(be2_ref[...], xgT,
                   preferred_element_type=jnp.float32)
    cols = []
    for b in range(BE // 128):
        lo, hi = b * 128, (b + 1) * 128
        acc = bias[:, lo:hi]
        for i in range(DIM):
            acc = acc + (ewT[DIM * i:DIM * (i + 1), lo:hi]
                         * xgT[i:i + 1, lo:hi])
        cols.append(acc)
    msg_ref[...] = jnp.concatenate(cols, axis=1).T


_msg = pl.pallas_call(
    _msg_body,
    grid=(E_PAD // BE,),
    in_specs=[
        pl.BlockSpec((16, BE), lambda i: (0, i)),
        pl.BlockSpec((BE, DIM), lambda i: (i, 0)),
        pl.BlockSpec((F_IN, 16), lambda i: (0, 0)),
        pl.BlockSpec((F_IN, 1), lambda i: (0, 0)),
        pl.BlockSpec((DIM * DIM, F_IN), lambda i: (0, 0)),
        pl.BlockSpec((DIM, DIM), lambda i: (0, 0)),
    ],
    out_specs=pl.BlockSpec((BE, DIM), lambda i: (i, 0)),
    out_shape=jax.ShapeDtypeStruct((E_PAD, DIM), jnp.float32),
)


def _gru_body(agg_ref, cnt_ref, h_ref, bconv_ref, wih_ref, whh_ref,
              bih_ref, bhh_ref, out_ref):
    agg = agg_ref[0] + agg_ref[1]
    cnt = jnp.maximum(cnt_ref[0, :, 0:1] + cnt_ref[1, :, 0:1], 1.0)
    m = jnp.maximum(agg / cnt + bconv_ref[...], 0.0)
    h = h_ref[...]
    gi = jnp.dot(m, wih_ref[...], preferred_element_type=jnp.float32)
    gi = gi + bih_ref[...]
    gh = jnp.dot(h, whh_ref[...], preferred_element_type=jnp.float32)
    gh = gh + bhh_ref[...]
    r = jax.nn.sigmoid(gi[:, 0:DIM] + gh[:, 0:DIM])
    z = jax.nn.sigmoid(gi[:, DIM:2 * DIM] + gh[:, DIM:2 * DIM])
    n = jnp.tanh(gi[:, 2 * DIM:] + r * gh[:, 2 * DIM:])
    out_ref[...] = (1.0 - z) * n + z * h


_gru = pl.pallas_call(
    _gru_body,
    grid=(N_PAD // BN,),
    in_specs=[
        pl.BlockSpec((2, BN, DIM), lambda i: (0, i, 0)),
        pl.BlockSpec((2, BN, 16), lambda i: (0, i, 0)),
        pl.BlockSpec((BN, DIM), lambda i: (i, 0)),
        pl.BlockSpec((1, DIM), lambda i: (0, 0)),
        pl.BlockSpec((DIM, 3 * DIM), lambda i: (0, 0)),
        pl.BlockSpec((DIM, 3 * DIM), lambda i: (0, 0)),
        pl.BlockSpec((1, 3 * DIM), lambda i: (0, 0)),
        pl.BlockSpec((1, 3 * DIM), lambda i: (0, 0)),
    ],
    out_specs=pl.BlockSpec((BN, DIM), lambda i: (i, 0)),
    out_shape=jax.ShapeDtypeStruct((N_PAD, DIM), jnp.float32),
)


def _s2s_body(h_ref, gcol_ref, grow_ref, wih_ref, whh_ref, bih_ref, bhh_ref,
              q_ref):
    hfull = h_ref[...]
    gcol = gcol_ref[...]
    grow = grow_ref[...]
    iota_row = lax.broadcasted_iota(jnp.int32, (1, G), 1)
    onehot = (gcol == iota_row).astype(jnp.float32)
    iota_col = lax.broadcasted_iota(jnp.int32, (G, 1), 0)
    onehotT = (iota_col == grow).astype(jnp.float32)
    valid = (gcol < G).astype(jnp.float32)

    q_star = jnp.zeros((G, 2 * DIM), jnp.float32)
    lh = jnp.zeros((G, DIM), jnp.float32)
    lc = jnp.zeros((G, DIM), jnp.float32)
    for _ in range(3):
        gates = (jnp.dot(q_star, wih_ref[...], preferred_element_type=jnp.float32)
                 + bih_ref[...]
                 + jnp.dot(lh, whh_ref[...], preferred_element_type=jnp.float32)
                 + bhh_ref[...])
        i_ = jax.nn.sigmoid(gates[:, 0:DIM])
        f_ = jax.nn.sigmoid(gates[:, DIM:2 * DIM])
        g_ = jnp.tanh(gates[:, 2 * DIM:3 * DIM])
        o_ = jax.nn.sigmoid(gates[:, 3 * DIM:])
        lc = f_ * lc + i_ * g_
        lh = o_ * jnp.tanh(lc)
        q = lh
        qn = jnp.dot(onehot, q, preferred_element_type=jnp.float32)
        e = jnp.sum(hfull * qn, axis=1, keepdims=True)
        e = e * valid
        big = jnp.where(onehot > 0.0, e, -1e30)
        emax = jnp.max(big, axis=0, keepdims=True)
        emax = jnp.where(emax < -1e29, 0.0, emax)
        emax_n = jnp.sum(onehot * emax, axis=1, keepdims=True)
        ee = jnp.exp(e - emax_n) * valid
        denom = jnp.sum(onehot * ee, axis=0, keepdims=True)
        denom_n = jnp.sum(onehot * denom, axis=1, keepdims=True)
        a = ee / (denom_n + 1e-16)
        rvec = jnp.dot(onehotT, a * hfull,
                       preferred_element_type=jnp.float32)
        q_star = jnp.concatenate([q, rvec], axis=1)
    q_ref[...] = q_star


_s2s = pl.pallas_call(
    _s2s_body,
    out_shape=jax.ShapeDtypeStruct((G, 2 * DIM), jnp.float32),
)


def kernel(node_features, edge_features, W0, b0, We1, be1, We2, be2, b_conv,
           gW_ih, gW_hh, gb_ih, gb_hh, lW_ih, lW_hh, lb_ih, lb_hh,
           edge_index, graph_index):
    f32 = jnp.float32
    nf = jnp.zeros((N_PAD, F_IN), f32).at[:N].set(node_features)
    efT = jnp.zeros((16, E_PAD), f32).at[:11, :E].set(edge_features.T)
    src3 = (jnp.zeros((E_PAD,), jnp.int32).at[:E].set(edge_index[0])
            .reshape(NW, CHUNKS, CHUNK))
    dst3 = (jnp.full((E_PAD,), N, jnp.int32).at[:E].set(edge_index[1])
            .reshape(NW, CHUNKS, CHUNK))
    gcol = jnp.full((N_PAD, 1), G, jnp.int32).at[:N, 0].set(graph_index)
    grow = gcol.reshape(1, N_PAD)

    W0T = W0.T
    We1p = jnp.zeros((F_IN, 16), f32).at[:, :11].set(We1)
    be1c = be1.reshape(F_IN, 1)
    We2bf = We2.astype(jnp.bfloat16)
    be2m = be2.reshape(DIM, DIM).T
    b0r = b0.reshape(1, DIM)
    bconv = b_conv.reshape(1, DIM)
    gWihT = gW_ih.T
    gWhhT = gW_hh.T
    gbih = gb_ih.reshape(1, 3 * DIM)
    gbhh = gb_hh.reshape(1, 3 * DIM)
    lWihT = lW_ih.T
    lWhhT = lW_hh.T
    lbih = lb_ih.reshape(1, 4 * DIM)
    lbhh = lb_hh.reshape(1, 4 * DIM)

    zeros32 = jnp.zeros((STRIPE, DIM), f32)
    zeros16 = jnp.zeros((STRIPE, 16), f32)
    ones16 = jnp.ones((CHUNK, 16), f32)

    h = _lin0(nf, W0T, b0r)
    cntp = _sc_cnt(dst3, zeros16, ones16)
    for _ in range(3):
        xg = _sc_gather(h, src3)
        msg = _msg(efT, xg, We1p, be1c, We2bf, be2m)
        aggp = _sc_scatter(msg, dst3, zeros32)
        h = _gru(aggp, cntp, h, bconv, gWihT, gWhhT, gbih, gbhh)
    q_star = _s2s(h, gcol, grow, lWihT, lWhhT, lbih, lbhh)
    return (q_star, h[:N])


# speedup vs baseline: 4.0135x; 1.0030x over previous
import functools

import jax
import jax.numpy as jnp
from jax import lax
from jax.experimental import pallas as pl
from jax.experimental.pallas import tpu as pltpu
from jax.experimental.pallas import tpu_sc as plsc

N = 10000
E = 160000
F_IN = 128
DIM = 32
G = 64

NW = 32
CHUNK = 128
CHUNKS = 40
PER_W = CHUNKS * CHUNK
E_PAD = NW * PER_W
N_PAD = 10240
STRIPE = N_PAD // 16
BE = 4096
BN = 1024

_MESH = plsc.VectorSubcoreMesh(
    core_axis_name="c", subcore_axis_name="s", num_cores=2, num_subcores=16)


NBUF = 8
ROUNDS = CHUNKS // NBUF


@functools.partial(
    pl.kernel,
    out_type=jax.ShapeDtypeStruct((E_PAD, DIM), jnp.float32),
    mesh=_MESH,
    scratch_types=[
        pltpu.VMEM((CHUNKS, CHUNK), jnp.int32),
        pltpu.VMEM((NBUF, CHUNK, DIM), jnp.float32),
        pltpu.VMEM_SHARED((N_PAD, DIM), jnp.float32),
        pltpu.SemaphoreType.DMA,
        pltpu.SemaphoreType.DMA,
    ],
    compiler_params=pltpu.CompilerParams(use_tc_tiling_on_sc=False),
)
def _sc_gather(table, src3, out, idx_v, bufs, tbl, gsem, wsem):
    cid = lax.axis_index("c")
    sid = lax.axis_index("s")
    w = sid * 2 + cid
    r0 = sid * STRIPE
    pltpu.sync_copy(table.at[pl.ds(r0, STRIPE)], tbl.at[pl.ds(r0, STRIPE)])
    pltpu.sync_copy(src3.at[w], idx_v)
    plsc.subcore_barrier()

    def round_(i, carry):
        @pl.when(i > 0)
        def _():
            for k in range(NBUF):
                pltpu.make_async_copy(
                    bufs.at[k], out.at[pl.ds(w * PER_W, CHUNK)], wsem).wait()
        gds = [
            pltpu.async_copy(tbl.at[idx_v.at[i * NBUF + k]], bufs.at[k], gsem)
            for k in range(NBUF)
        ]
        for k in range(NBUF):
            gds[k].wait()
        for k in range(NBUF):
            pltpu.async_copy(
                bufs.at[k],
                out.at[pl.ds(w * PER_W + (i * NBUF + k) * CHUNK, CHUNK)],
                wsem)
        return carry

    lax.fori_loop(0, ROUNDS, round_, 0)
    for k in range(NBUF):
        pltpu.make_async_copy(
            bufs.at[k], out.at[pl.ds(w * PER_W, CHUNK)], wsem).wait()


@functools.partial(
    pl.kernel,
    out_type=jax.ShapeDtypeStruct((2, N_PAD, DIM), jnp.float32),
    mesh=_MESH,
    scratch_types=[
        pltpu.VMEM((CHUNKS, CHUNK), jnp.int32),
        pltpu.VMEM((2 * NBUF, CHUNK, DIM), jnp.float32),
        pltpu.VMEM_SHARED((N_PAD, DIM), jnp.float32),
        pltpu.SemaphoreType.DMA,
    ],
    compiler_params=pltpu.CompilerParams(use_tc_tiling_on_sc=False),
)
def _sc_scatter(msg, dst3, zeros32, agg_out, idx_v, bufs, acc, lsem):
    cid = lax.axis_index("c")
    sid = lax.axis_index("s")
    w = sid * 2 + cid
    r0 = sid * STRIPE
    pltpu.sync_copy(zeros32, acc.at[pl.ds(r0, STRIPE)])
    pltpu.sync_copy(dst3.at[w], idx_v)
    for k in range(NBUF):
        pltpu.async_copy(msg.at[pl.ds((w * CHUNKS + k) * CHUNK, CHUNK)],
                         bufs.at[k], lsem)
    plsc.subcore_barrier()

    def round_(i, carry):
        p = lax.rem(i, 2)
        for k in range(NBUF):
            pltpu.make_async_copy(
                msg.at[pl.ds(w * PER_W, CHUNK)], bufs.at[k], lsem).wait()
        @pl.when(i < ROUNDS - 1)
        def _():
            for k in range(NBUF):
                pltpu.async_copy(
                    msg.at[pl.ds((w * CHUNKS + (i + 1) * NBUF + k) * CHUNK,
                                 CHUNK)],
                    bufs.at[(1 - p) * NBUF + k], lsem)
        for k in range(NBUF):
            pltpu.sync_copy(bufs.at[p * NBUF + k],
                            acc.at[idx_v.at[i * NBUF + k]], add=True)
        return carry

    lax.fori_loop(0, ROUNDS, round_, 0)
    plsc.subcore_barrier()
    pltpu.sync_copy(acc.at[pl.ds(r0, STRIPE)],
                    agg_out.at[cid, pl.ds(r0, STRIPE)])


@functools.partial(
    pl.kernel,
    out_type=jax.ShapeDtypeStruct((2, N_PAD, 16), jnp.float32),
    mesh=_MESH,
    scratch_types=[
        pltpu.VMEM((CHUNKS, CHUNK), jnp.int32),
        pltpu.VMEM((CHUNK, 16), jnp.float32),
        pltpu.VMEM_SHARED((N_PAD, 16), jnp.float32),
    ],
    compiler_params=pltpu.CompilerParams(use_tc_tiling_on_sc=False),
)
def _sc_cnt(dst3, zeros16, ones16, cnt_out, idx_v, ones_v, cacc):
    cid = lax.axis_index("c")
    sid = lax.axis_index("s")
    w = sid * 2 + cid
    r0 = sid * STRIPE
    pltpu.sync_copy(zeros16, cacc.at[pl.ds(r0, STRIPE)])
    pltpu.sync_copy(ones16, ones_v)
    pltpu.sync_copy(dst3.at[w], idx_v)
    plsc.subcore_barrier()

    def step(j, carry):
        pltpu.sync_copy(ones_v, cacc.at[idx_v.at[j]], add=True)
        return carry

    lax.fori_loop(0, CHUNKS, step, 0)
    plsc.subcore_barrier()
    pltpu.sync_copy(cacc.at[pl.ds(r0, STRIPE)],
                    cnt_out.at[cid, pl.ds(r0, STRIPE)])


def _lin0_body(nf_ref, w_ref, b_ref, out_ref):
    acc = jnp.dot(nf_ref[...], w_ref[...], preferred_element_type=jnp.float32)
    out_ref[...] = jnp.maximum(acc + b_ref[...], 0.0)


_lin0 = pl.pallas_call(
    _lin0_body,
    grid=(N_PAD // BN,),
    in_specs=[
        pl.BlockSpec((BN, F_IN), lambda i: (i, 0)),
        pl.BlockSpec((F_IN, DIM), lambda i: (0, 0)),
        pl.BlockSpec((1, DIM), lambda i: (0, 0)),
    ],
    out_specs=pl.BlockSpec((BN, DIM), lambda i: (i, 0)),
    out_shape=jax.ShapeDtypeStruct((N_PAD, DIM), jnp.float32),
)


def _msg_body(efT_ref, xg_ref, we1_ref, be1_ref, we2_ref, be2_ref, msg_ref):
    h1T = jnp.dot(we1_ref[...], efT_ref[...],
                  preferred_element_type=jnp.float32)
    h1T = jnp.maximum(h1T + be1_ref[...], 0.0)
    ewT = jnp.dot(we2_ref[...], h1T.astype(jnp.bfloat16),
                  preferred_element_type=jnp.float32)
    xgT = xg_ref[...].T
    bias = jnp.dot(be2_ref[...], xgT,
                   preferred_element_type=jnp.float32)
    cols = []
    for b in range(BE // 128):
        lo, hi = b * 128, (b + 1) * 128
        acc = bias[:, lo:hi]
        for i in range(DIM):
            acc = acc + (ewT[DIM * i:DIM * (i + 1), lo:hi]
                         * xgT[i:i + 1, lo:hi])
        cols.append(acc)
    msg_ref[...] = jnp.concatenate(cols, axis=1).T


_msg = pl.pallas_call(
    _msg_body,
    grid=(E_PAD // BE,),
    in_specs=[
        pl.BlockSpec((16, BE), lambda i: (0, i)),
        pl.BlockSpec((BE, DIM), lambda i: (i, 0)),
        pl.BlockSpec((F_IN, 16), lambda i: (0, 0)),
        pl.BlockSpec((F_IN, 1), lambda i: (0, 0)),
        pl.BlockSpec((DIM * DIM, F_IN), lambda i: (0, 0)),
        pl.BlockSpec((DIM, DIM), lambda i: (0, 0)),
    ],
    out_specs=pl.BlockSpec((BE, DIM), lambda i: (i, 0)),
    out_shape=jax.ShapeDtypeStruct((E_PAD, DIM), jnp.float32),
)


def _gru_body(agg_ref, cnt_ref, h_ref, bconv_ref, wih_ref, whh_ref,
              bih_ref, bhh_ref, out_ref):
    agg = agg_ref[0] + agg_ref[1]
    cnt = jnp.maximum(cnt_ref[0, :, 0:1] + cnt_ref[1, :, 0:1], 1.0)
    m = jnp.maximum(agg / cnt + bconv_ref[...], 0.0)
    h = h_ref[...]
    gi = jnp.dot(m, wih_ref[...], preferred_element_type=jnp.float32)
    gi = gi + bih_ref[...]
    gh = jnp.dot(h, whh_ref[...], preferred_element_type=jnp.float32)
    gh = gh + bhh_ref[...]
    r = jax.nn.sigmoid(gi[:, 0:DIM] + gh[:, 0:DIM])
    z = jax.nn.sigmoid(gi[:, DIM:2 * DIM] + gh[:, DIM:2 * DIM])
    n = jnp.tanh(gi[:, 2 * DIM:] + r * gh[:, 2 * DIM:])
    out_ref[...] = (1.0 - z) * n + z * h


_gru = pl.pallas_call(
    _gru_body,
    grid=(N_PAD // BN,),
    in_specs=[
        pl.BlockSpec((2, BN, DIM), lambda i: (0, i, 0)),
        pl.BlockSpec((2, BN, 16), lambda i: (0, i, 0)),
        pl.BlockSpec((BN, DIM), lambda i: (i, 0)),
        pl.BlockSpec((1, DIM), lambda i: (0, 0)),
        pl.BlockSpec((DIM, 3 * DIM), lambda i: (0, 0)),
        pl.BlockSpec((DIM, 3 * DIM), lambda i: (0, 0)),
        pl.BlockSpec((1, 3 * DIM), lambda i: (0, 0)),
        pl.BlockSpec((1, 3 * DIM), lambda i: (0, 0)),
    ],
    out_specs=pl.BlockSpec((BN, DIM), lambda i: (i, 0)),
    out_shape=jax.ShapeDtypeStruct((N_PAD, DIM), jnp.float32),
)


def _s2s_body(h_ref, gcol_ref, grow_ref, wih_ref, whh_ref, bih_ref, bhh_ref,
              q_ref):
    hfull = h_ref[...]
    gcol = gcol_ref[...]
    grow = grow_ref[...]
    iota_row = lax.broadcasted_iota(jnp.int32, (1, G), 1)
    onehot = (gcol == iota_row).astype(jnp.float32)
    iota_col = lax.broadcasted_iota(jnp.int32, (G, 1), 0)
    onehotT = (iota_col == grow).astype(jnp.float32)
    valid = (gcol < G).astype(jnp.float32)

    q_star = jnp.zeros((G, 2 * DIM), jnp.float32)
    lh = jnp.zeros((G, DIM), jnp.float32)
    lc = jnp.zeros((G, DIM), jnp.float32)
    for _ in range(3):
        gates = (jnp.dot(q_star, wih_ref[...], preferred_element_type=jnp.float32)
                 + bih_ref[...]
                 + jnp.dot(lh, whh_ref[...], preferred_element_type=jnp.float32)
                 + bhh_ref[...])
        i_ = jax.nn.sigmoid(gates[:, 0:DIM])
        f_ = jax.nn.sigmoid(gates[:, DIM:2 * DIM])
        g_ = jnp.tanh(gates[:, 2 * DIM:3 * DIM])
        o_ = jax.nn.sigmoid(gates[:, 3 * DIM:])
        lc = f_ * lc + i_ * g_
        lh = o_ * jnp.tanh(lc)
        q = lh
        qn = jnp.dot(onehot, q, preferred_element_type=jnp.float32)
        e = jnp.sum(hfull * qn, axis=1, keepdims=True)
        e = e * valid
        big = jnp.where(onehot > 0.0, e, -1e30)
        emax = jnp.max(big, axis=0, keepdims=True)
        emax = jnp.where(emax < -1e29, 0.0, emax)
        emax_n = jnp.sum(onehot * emax, axis=1, keepdims=True)
        ee = jnp.exp(e - emax_n) * valid
        denom = jnp.sum(onehot * ee, axis=0, keepdims=True)
        denom_n = jnp.sum(onehot * denom, axis=1, keepdims=True)
        a = ee / (denom_n + 1e-16)
        rvec = jnp.dot(onehotT, a * hfull,
                       preferred_element_type=jnp.float32)
        q_star = jnp.concatenate([q, rvec], axis=1)
    q_ref[...] = q_star


_s2s = pl.pallas_call(
    _s2s_body,
    out_shape=jax.ShapeDtypeStruct((G, 2 * DIM), jnp.float32),
)


def kernel(node_features, edge_features, W0, b0, We1, be1, We2, be2, b_conv,
           gW_ih, gW_hh, gb_ih, gb_hh, lW_ih, lW_hh, lb_ih, lb_hh,
           edge_index, graph_index):
    f32 = jnp.float32
    nf = jnp.zeros((N_PAD, F_IN), f32).at[:N].set(node_features)
    efT = jnp.zeros((16, E_PAD), f32).at[:11, :E].set(edge_features.T)
    src3 = (jnp.zeros((E_PAD,), jnp.int32).at[:E].set(edge_index[0])
            .reshape(NW, CHUNKS, CHUNK))
    dst3 = (jnp.full((E_PAD,), N, jnp.int32).at[:E].set(edge_index[1])
            .reshape(NW, CHUNKS, CHUNK))
    gcol = jnp.full((N_PAD, 1), G, jnp.int32).at[:N, 0].set(graph_index)
    grow = gcol.reshape(1, N_PAD)

    W0T = W0.T
    We1p = jnp.zeros((F_IN, 16), f32).at[:, :11].set(We1)
    be1c = be1.reshape(F_IN, 1)
    We2bf = We2.astype(jnp.bfloat16)
    be2m = be2.reshape(DIM, DIM).T
    b0r = b0.reshape(1, DIM)
    bconv = b_conv.reshape(1, DIM)
    gWihT = gW_ih.T
    gWhhT = gW_hh.T
    gbih = gb_ih.reshape(1, 3 * DIM)
    gbhh = gb_hh.reshape(1, 3 * DIM)
    lWihT = lW_ih.T
    lWhhT = lW_hh.T
    lbih = lb_ih.reshape(1, 4 * DIM)
    lbhh = lb_hh.reshape(1, 4 * DIM)

    zeros32 = jnp.zeros((STRIPE, DIM), f32)
    zeros16 = jnp.zeros((STRIPE, 16), f32)
    ones16 = jnp.ones((CHUNK, 16), f32)

    h = _lin0(nf, W0T, b0r)
    cntp = _sc_cnt(dst3, zeros16, ones16)
    for _ in range(3):
        xg = _sc_gather(h, src3)
        msg = _msg(efT, xg, We1p, be1c, We2bf, be2m)
        aggp = _sc_scatter(msg, dst3, zeros32)
        h = _gru(aggp, cntp, h, bconv, gWihT, gWhhT, gbih, gbhh)
    q_star = _s2s(h, gcol, grow, lWihT, lWhhT, lbih, lbhh)
    return (q_star, h[:N])
